# baseline (speedup 1.0000x reference)
import jax
import jax.numpy as jnp
from jax.experimental import pallas as pl

N = 50000
H = 2
D = 64
CH = 2
C = 256


def _mm_kernel(x_ref, w_ref, o_ref):
    o_ref[...] = jnp.dot(x_ref[...], w_ref[...], preferred_element_type=jnp.float32)


def _mm(x, w):
    n, k = x.shape
    m = w.shape[1]
    blk = 2000
    assert n % blk == 0
    return pl.pallas_call(
        _mm_kernel,
        grid=(n // blk,),
        in_specs=[pl.BlockSpec((blk, k), lambda i: (i, 0)),
                  pl.BlockSpec((k, m), lambda i: (0, 0))],
        out_specs=pl.BlockSpec((blk, m), lambda i: (i, 0)),
        out_shape=jax.ShapeDtypeStruct((n, m), jnp.float32),
    )(x, w)


def _edge_softmax(logits, dst, n):
    m = jax.ops.segment_max(logits, dst, num_segments=n)
    m = jnp.where(jnp.isfinite(m), m, 0.0)
    ex = jnp.exp(logits - m[dst])
    denom = jax.ops.segment_sum(ex, dst, num_segments=n)
    return ex / (denom[dst] + 1e-16)


def _gatv2(h, src, dst, W_src, W_dst, attn, bias, n, heads, dim):
    fs = _mm(h, W_src).reshape(n, heads, dim)
    fd = _mm(h, W_dst).reshape(n, heads, dim)
    e = jax.nn.leaky_relu(fs[src] + fd[dst], 0.2)
    logits = (e * attn[None]).sum(-1)
    alpha = _edge_softmax(logits, dst, n)
    out = jax.ops.segment_sum(fs[src] * alpha[..., None], dst, num_segments=n)
    return out + bias[None]


def _gat(h, src, dst, W, attn_l, attn_r, bias, n, heads, dim):
    f = _mm(h, W).reshape(n, heads, dim)
    el = (f * attn_l[None]).sum(-1)
    er = (f * attn_r[None]).sum(-1)
    logits = jax.nn.leaky_relu(el[src] + er[dst], 0.2)
    alpha = _edge_softmax(logits, dst, n)
    out = jax.ops.segment_sum(f[src] * alpha[..., None], dst, num_segments=n)
    return out + bias[None]


def kernel(in_feat, edge_index, emb, W_src1, W_dst1, attn1, bias1, W_src2, W_dst2,
           attn2, bias2, Wc, attn_lc, attn_rc, biasc):
    src = edge_index[0]
    dst = edge_index[1]
    h = emb[in_feat]
    h = _gatv2(h, src, dst, W_src1, W_dst1, attn1, bias1, N, H, D).reshape(N, -1)
    h = jax.nn.elu(h)
    h = _gatv2(h, src, dst, W_src2, W_dst2, attn2, bias2, N, H, D).reshape(N, -1)
    h = jax.nn.elu(h)
    h = _gat(h, src, dst, Wc, attn_lc, attn_rc, biasc, N, CH, C)
    h = h.mean(axis=1)
    return jax.nn.softmax(h, axis=1)


# speedup vs baseline: 28.2936x; 28.2936x over previous
import functools

import jax
import jax.numpy as jnp
from jax import lax
from jax.experimental import pallas as pl
from jax.experimental.pallas import tpu as pltpu
from jax.experimental.pallas import tpu_sc as plsc

N = 50000
E = 800000
VOCAB = 256
EMB = 64

NR = 8192
RPC = 4
NPAD = NR * RPC * 2
NR3 = 3072
RPC3 = 9
NPAD3 = NR3 * RPC3 * 2
CB = 1024
NCHUNK = 49
EPAD = 16 * NCHUNK * CB
B = 128
HB = 1280
ZR = 16


def _i16(v):
    return jnp.full((16,), v, jnp.int32)


def _mo(x, m=8):
    return pl.multiple_of(x, m)


def _make_edge_v2():
    mesh = plsc.VectorSubcoreMesh(core_axis_name="c", subcore_axis_name="s")

    @functools.partial(
        pl.kernel,
        mesh=mesh,
        compiler_params=pltpu.CompilerParams(needs_layout_passes=False),
        out_type=[
            jax.ShapeDtypeStruct((NPAD, 128), jnp.float32),
            jax.ShapeDtypeStruct((NPAD * 2 // 128, 128), jnp.float32),
        ],
        scratch_types=[
            pltpu.VMEM_SHARED((NR + 8, 128), jnp.float32),
            pltpu.VMEM_SHARED((128, 128), jnp.float32),
            pltpu.VMEM((CB,), jnp.int32),
            pltpu.VMEM((CB,), jnp.int32),
            pltpu.VMEM((HB,), jnp.int32),
            pltpu.VMEM((HB,), jnp.int32),
            pltpu.VMEM((B,), jnp.int32),
            pltpu.VMEM((B,), jnp.int32),
            pltpu.VMEM((B,), jnp.int32),
            pltpu.VMEM((B + 16,), jnp.int32),
            pltpu.VMEM((B,), jnp.int32),
            pltpu.VMEM((B, 128), jnp.float32),
            pltpu.VMEM((B, 128), jnp.float32),
            pltpu.VMEM((136, 128), jnp.float32),
            pltpu.VMEM((ZR, 128), jnp.float32),
            pltpu.VMEM((2, 64), jnp.float32),
            pltpu.SemaphoreType.DMA,
            pltpu.SemaphoreType.DMA,
        ],
    )
    def k(fs_hbm, fd_hbm, src_hbm, dst_hbm, attn_hbm, acc_hbm, den_hbm,
          acc_sp, den_sp, sbuf, dbuf, hs, hd, gsrc, gdst, locb, locpad,
          ident, fsrow, fdrow, den_tile, zblk, attn_v, sem, sem2):
        c = lax.axis_index("c")
        s = lax.axis_index("s")
        iota16 = lax.iota(jnp.int32, 16)
        zeros16 = jnp.zeros((16,), jnp.float32)
        ebase = s * (EPAD // 16)
        lane0 = iota16 == 0

        pltpu.sync_copy(attn_hbm, attn_v)

        def zrow(r, _):
            for kk in range(8):
                zblk[r, pl.ds(kk * 16, 16)] = zeros16
            return 0
        lax.fori_loop(0, ZR, zrow, 0)
        for g in range(8):
            ident[pl.ds(g * 16, 16)] = iota16 + g * 16

        def process_block(off, valid, lo):
            def bg(g, _):
                dv = hd[pl.ds(off + g * 16, 16)]
                sv = hs[pl.ds(off + g * 16, 16)]
                m = (g * 16 + iota16) < valid
                lc = jnp.where(m, dv - lo, NR)
                gdst[pl.ds(g * 16, 16)] = jnp.where(m, dv, 0)
                gsrc[pl.ds(g * 16, 16)] = jnp.where(m, sv, 0)
                locb[pl.ds(g * 16, 16)] = lc
                locpad[pl.ds(g * 16, 16)] = lc
                return 0
            lax.fori_loop(0, B // 16, bg, 0)
            cp1 = pltpu.async_copy(fs_hbm.at[gsrc], fsrow, sem)
            cp2 = pltpu.async_copy(fd_hbm.at[gdst], fdrow, sem2)
            cp1.wait()
            cp2.wait()
            av = tuple(attn_v[hh, pl.ds(kk * 16, 16)]
                       for hh in range(2) for kk in range(4))

            def pedge(p, carry):
                x = [fsrow[p, pl.ds(kk * 16, 16)] for kk in range(8)]
                y = [fdrow[p, pl.ds(kk * 16, 16)] for kk in range(8)]
                w = []
                for kk in range(8):
                    t = x[kk] + y[kk]
                    w.append(jnp.maximum(t, 0.2 * t) * carry[kk])
                l0 = jnp.sum((w[0] + w[1]) + (w[2] + w[3]))
                l1 = jnp.sum((w[4] + w[5]) + (w[6] + w[7]))
                ex0 = jnp.exp(jnp.full((16,), l0, jnp.float32))
                ex1 = jnp.exp(jnp.full((16,), l1, jnp.float32))
                for kk in range(4):
                    fsrow[p, pl.ds(kk * 16, 16)] = x[kk] * ex0
                for kk in range(4, 8):
                    fsrow[p, pl.ds(kk * 16, 16)] = x[kk] * ex1
                ls = locpad[pl.ds(p, 16)][_i16(0)]
                dm = ls == NR
                d0 = jnp.where(dm, _i16(2 * NR), 2 * ls)
                d1 = jnp.where(dm, _i16(2 * NR), 2 * ls + 1)
                plsc.addupdate_scatter(
                    den_tile, [d0 >> 7, d0 & 127], ex0, mask=lane0)
                plsc.addupdate_scatter(
                    den_tile, [d1 >> 7, d1 & 127], ex1, mask=lane0)
                return carry
            lax.fori_loop(0, B, pedge, av)
            pltpu.sync_copy(fsrow, acc_sp.at[locb], add=True)

        def sweep(r, _):
            lo = (c * RPC + r) * NR

            def zcp(j, _):
                row0 = _mo(s * (NR // 16) + j * ZR)
                pltpu.sync_copy(zblk, acc_sp.at[pl.ds(row0, ZR), :])
                return 0
            lax.fori_loop(0, NR // 16 // ZR, zcp, 0)

            def zdt(j, _):
                for kk in range(8):
                    den_tile[j, pl.ds(kk * 16, 16)] = zeros16
                return 0
            lax.fori_loop(0, 136, zdt, 0)
            pltpu.sync_copy(zblk.at[pl.ds(0, 8), :],
                            den_sp.at[pl.ds(_mo(s * 8), 8), :])
            plsc.subcore_barrier()

            def chunk(kk, cnt):
                pltpu.sync_copy(src_hbm.at[pl.ds(_mo(ebase + kk * CB), CB)], sbuf)
                pltpu.sync_copy(dst_hbm.at[pl.ds(_mo(ebase + kk * CB), CB)], dbuf)

                def scan(g, cnt):
                    dv = dbuf[pl.ds(g * 16, 16)]
                    sv = sbuf[pl.ds(g * 16, 16)]
                    m = (dv >= lo) & (dv < lo + NR)
                    pos = plsc.cumsum(m.astype(jnp.int32)) - 1 + cnt
                    plsc.store_scatter(hd, [pos], dv, mask=m)
                    plsc.store_scatter(hs, [pos], sv, mask=m)
                    return cnt + plsc.all_reduce_population_count(m)
                cnt = lax.fori_loop(0, CB // 16, scan, cnt)
                navail = cnt[0]
                nblk = navail // B

                def pb(i, _):
                    process_block(i * B, B, lo)
                    return 0
                lax.fori_loop(0, nblk, pb, 0)
                rem = navail - nblk * B

                def mv(g, _):
                    @pl.when(g * 16 < rem)
                    def _():
                        hd[pl.ds(g * 16, 16)] = hd[pl.ds(nblk * B + g * 16, 16)]
                        hs[pl.ds(g * 16, 16)] = hs[pl.ds(nblk * B + g * 16, 16)]
                    return 0
                lax.fori_loop(0, B // 16, mv, 0)
                return _i16(0) + rem
            cnt = lax.fori_loop(0, NCHUNK, chunk, _i16(0))
            rem = cnt[0]

            @pl.when(rem > 0)
            def _():
                process_block(0, rem, lo)
            plsc.subcore_barrier()
            pltpu.sync_copy(den_tile.at[pl.ds(0, 128), :],
                            den_sp.at[ident], add=True)
            plsc.subcore_barrier()

            def fcp(j, _):
                row0 = _mo(s * (NR // 16) + j * ZR)
                pltpu.sync_copy(acc_sp.at[pl.ds(row0, ZR), :],
                                acc_hbm.at[pl.ds(_mo(lo + row0), ZR), :])
                return 0
            lax.fori_loop(0, NR // 16 // ZR, fcp, 0)

            @pl.when(s == 0)
            def _():
                pltpu.sync_copy(den_sp,
                                den_hbm.at[pl.ds(_mo(lo // 64), 128), :])
            plsc.subcore_barrier()
            return 0
        lax.fori_loop(0, RPC, sweep, 0)

    return k


def _make_edge_gat():
    mesh = plsc.VectorSubcoreMesh(core_axis_name="c", subcore_axis_name="s")

    @functools.partial(
        pl.kernel,
        mesh=mesh,
        compiler_params=pltpu.CompilerParams(needs_layout_passes=False),
        out_type=[
            jax.ShapeDtypeStruct((NPAD3, 128), jnp.float32),
            jax.ShapeDtypeStruct((NPAD3, 128), jnp.float32),
            jax.ShapeDtypeStruct((NPAD3 * 2 // 128, 128), jnp.float32),
        ],
        scratch_types=[
            pltpu.VMEM_SHARED((NR3 + 8, 128), jnp.float32),
            pltpu.VMEM_SHARED((NR3 + 8, 128), jnp.float32),
            pltpu.VMEM_SHARED((128, 128), jnp.float32),
            pltpu.VMEM((CB,), jnp.int32),
            pltpu.VMEM((CB,), jnp.int32),
            pltpu.VMEM((HB,), jnp.int32),
            pltpu.VMEM((HB,), jnp.int32),
            pltpu.VMEM((B,), jnp.int32),
            pltpu.VMEM((B,), jnp.int32),
            pltpu.VMEM((B,), jnp.int32),
            pltpu.VMEM((B + 16,), jnp.int32),
            pltpu.VMEM((B,), jnp.int32),
            pltpu.VMEM((B + 16,), jnp.float32),
            pltpu.VMEM((B, 128), jnp.float32),
            pltpu.VMEM((B, 128), jnp.float32),
            pltpu.VMEM((B, 128), jnp.float32),
            pltpu.VMEM((136, 128), jnp.float32),
            pltpu.VMEM((ZR, 128), jnp.float32),
            pltpu.SemaphoreType.DMA,
            pltpu.SemaphoreType.DMA,
            pltpu.SemaphoreType.DMA,
        ],
    )
    def k(h2_hbm, elr_hbm, src_hbm, dst_hbm, acc0_hbm, acc1_hbm, den_hbm,
          acc0_sp, acc1_sp, den_sp, sbuf, dbuf, hs, hd, gsrc, gdst, locb,
          locpad, ident, exrb, h2row, elrow, errow, den_tile, zblk,
          sem, sem2, sem3):
        c = lax.axis_index("c")
        s = lax.axis_index("s")
        iota16 = lax.iota(jnp.int32, 16)
        zeros16 = jnp.zeros((16,), jnp.float32)
        ebase = s * (EPAD // 16)
        lane0 = iota16 == 0

        def zrow(r, _):
            for kk in range(8):
                zblk[r, pl.ds(kk * 16, 16)] = zeros16
            return 0
        lax.fori_loop(0, ZR, zrow, 0)
        for g in range(8):
            ident[pl.ds(g * 16, 16)] = iota16 + g * 16

        def process_block(off, valid, lo):
            def bg(g, _):
                dv = hd[pl.ds(off + g * 16, 16)]
                sv = hs[pl.ds(off + g * 16, 16)]
                m = (g * 16 + iota16) < valid
                lc = jnp.where(m, dv - lo, NR3)
                gdst[pl.ds(g * 16, 16)] = jnp.where(m, dv, 0)
                gsrc[pl.ds(g * 16, 16)] = jnp.where(m, sv, 0)
                locb[pl.ds(g * 16, 16)] = lc
                locpad[pl.ds(g * 16, 16)] = lc
                return 0
            lax.fori_loop(0, B // 16, bg, 0)
            cp1 = pltpu.async_copy(h2_hbm.at[gsrc], h2row, sem)
            cp2 = pltpu.async_copy(elr_hbm.at[gsrc], elrow, sem2)
            cp3 = pltpu.async_copy(elr_hbm.at[gdst], errow, sem3)
            cp1.wait()
            cp2.wait()
            cp3.wait()

            def pedge(p, _):
                evl = elrow[p, pl.ds(0, 16)]
                evr = errow[p, pl.ds(0, 16)]
                tv = evl + jnp.where(lane0, evr[_i16(2)], evr[_i16(3)])
                lv = jnp.maximum(tv, 0.2 * tv)
                ex0 = jnp.exp(lv[_i16(0)])
                ex1 = jnp.exp(lv[_i16(1)])
                for kk in range(8):
                    v = h2row[p, pl.ds(kk * 16, 16)]
                    h2row[p, pl.ds(kk * 16, 16)] = v * ex0
                plsc.store_scatter(exrb, [_i16(0) + p], ex1 / ex0, mask=lane0)
                ls = locpad[pl.ds(p, 16)][_i16(0)]
                dm = ls == NR3
                d0 = jnp.where(dm, _i16(2 * NR3), 2 * ls)
                d1 = jnp.where(dm, _i16(2 * NR3), 2 * ls + 1)
                plsc.addupdate_scatter(
                    den_tile, [d0 >> 7, d0 & 127], ex0, mask=lane0)
                plsc.addupdate_scatter(
                    den_tile, [d1 >> 7, d1 & 127], ex1, mask=lane0)
                return 0
            lax.fori_loop(0, B, pedge, 0)
            pltpu.sync_copy(h2row, acc0_sp.at[locb], add=True)

            def redge(p, _):
                rs = exrb[pl.ds(p, 16)][_i16(0)]
                for kk in range(8):
                    h2row[p, pl.ds(kk * 16, 16)] = (
                        h2row[p, pl.ds(kk * 16, 16)] * rs)
                return 0
            lax.fori_loop(0, B, redge, 0)
            pltpu.sync_copy(h2row, acc1_sp.at[locb], add=True)

        def sweep(r, _):
            lo = (c * RPC3 + r) * NR3

            def zcp(j, _):
                row0 = _mo(s * (NR3 // 16) + j * ZR)
                pltpu.sync_copy(zblk, acc0_sp.at[pl.ds(row0, ZR), :])
                pltpu.sync_copy(zblk, acc1_sp.at[pl.ds(row0, ZR), :])
                return 0
            lax.fori_loop(0, NR3 // 16 // ZR, zcp, 0)

            def zdt(j, _):
                for kk in range(8):
                    den_tile[j, pl.ds(kk * 16, 16)] = zeros16
                return 0
            lax.fori_loop(0, 136, zdt, 0)
            pltpu.sync_copy(zblk.at[pl.ds(0, 8), :],
                            den_sp.at[pl.ds(_mo(s * 8), 8), :])
            plsc.subcore_barrier()

            def chunk(kk, cnt):
                pltpu.sync_copy(src_hbm.at[pl.ds(_mo(ebase + kk * CB), CB)], sbuf)
                pltpu.sync_copy(dst_hbm.at[pl.ds(_mo(ebase + kk * CB), CB)], dbuf)

                def scan(g, cnt):
                    dv = dbuf[pl.ds(g * 16, 16)]
                    sv = sbuf[pl.ds(g * 16, 16)]
                    m = (dv >= lo) & (dv < lo + NR3)
                    pos = plsc.cumsum(m.astype(jnp.int32)) - 1 + cnt
                    plsc.store_scatter(hd, [pos], dv, mask=m)
                    plsc.store_scatter(hs, [pos], sv, mask=m)
                    return cnt + plsc.all_reduce_population_count(m)
                cnt = lax.fori_loop(0, CB // 16, scan, cnt)
                navail = cnt[0]
                nblk = navail // B

                def pb(i, _):
                    process_block(i * B, B, lo)
                    return 0
                lax.fori_loop(0, nblk, pb, 0)
                rem = navail - nblk * B

                def mv(g, _):
                    @pl.when(g * 16 < rem)
                    def _():
                        hd[pl.ds(g * 16, 16)] = hd[pl.ds(nblk * B + g * 16, 16)]
                        hs[pl.ds(g * 16, 16)] = hs[pl.ds(nblk * B + g * 16, 16)]
                    return 0
                lax.fori_loop(0, B // 16, mv, 0)
                return _i16(0) + rem
            cnt = lax.fori_loop(0, NCHUNK, chunk, _i16(0))
            rem = cnt[0]

            @pl.when(rem > 0)
            def _():
                process_block(0, rem, lo)
            plsc.subcore_barrier()
            pltpu.sync_copy(den_tile.at[pl.ds(0, 128), :],
                            den_sp.at[ident], add=True)
            plsc.subcore_barrier()

            def fcp(j, _):
                row0 = _mo(s * (NR3 // 16) + j * ZR)
                pltpu.sync_copy(acc0_sp.at[pl.ds(row0, ZR), :],
                                acc0_hbm.at[pl.ds(_mo(lo + row0), ZR), :])
                pltpu.sync_copy(acc1_sp.at[pl.ds(row0, ZR), :],
                                acc1_hbm.at[pl.ds(_mo(lo + row0), ZR), :])
                return 0
            lax.fori_loop(0, NR3 // 16 // ZR, fcp, 0)

            @pl.when(s == 0)
            def _():
                pltpu.sync_copy(den_sp.at[pl.ds(0, 48), :],
                                den_hbm.at[pl.ds(_mo(lo // 64), 48), :])
            plsc.subcore_barrier()
            return 0
        lax.fori_loop(0, RPC3, sweep, 0)

    return k


_edge_v2 = _make_edge_v2()
_edge_gat = _make_edge_gat()


def _prep(in_feat3, emb, Ws, Wd):
    def body(if_ref, emb_ref, ws_ref, wd_ref, fs_ref, fd_ref):
        ids = if_ref[0, 0, :]
        oh = (ids[:, None] == lax.broadcasted_iota(jnp.int32, (256, VOCAB), 1)
              ).astype(jnp.float32)
        h = jnp.dot(oh, emb_ref[...], preferred_element_type=jnp.float32)
        fs_ref[...] = jnp.dot(h, ws_ref[...], preferred_element_type=jnp.float32)
        fd_ref[...] = jnp.dot(h, wd_ref[...], preferred_element_type=jnp.float32)

    return pl.pallas_call(
        body,
        grid=(NPAD // 256,),
        in_specs=[
            pl.BlockSpec((1, 1, 256), lambda i: (i, 0, 0)),
            pl.BlockSpec((VOCAB, EMB), lambda i: (0, 0)),
            pl.BlockSpec((EMB, 128), lambda i: (0, 0)),
            pl.BlockSpec((EMB, 128), lambda i: (0, 0)),
        ],
        out_specs=[
            pl.BlockSpec((256, 128), lambda i: (i, 0)),
            pl.BlockSpec((256, 128), lambda i: (i, 0)),
        ],
        out_shape=[jax.ShapeDtypeStruct((NPAD, 128), jnp.float32)] * 2,
    )(in_feat3, emb, Ws, Wd)


def _normalize(a_ref, d_ref, b_ref):
    den0 = d_ref[:, 0:1] + 1e-16
    den1 = d_ref[:, 1:2] + 1e-16
    h = jnp.concatenate([a_ref[:, :64] / den0, a_ref[:, 64:] / den1], axis=1)
    h = h + b_ref[...]
    return jnp.where(h > 0, h, jnp.exp(jnp.minimum(h, 0.0)) - 1.0)


def _mid(acc, den, bias_flat, Wa, Wb):
    blk = 1024

    def body(a_ref, d_ref, b_ref, wa_ref, wb_ref, fs_ref, fd_ref):
        h = _normalize(a_ref, d_ref, b_ref)
        fs_ref[...] = jnp.dot(h, wa_ref[...], preferred_element_type=jnp.float32)
        fd_ref[...] = jnp.dot(h, wb_ref[...], preferred_element_type=jnp.float32)

    return pl.pallas_call(
        body,
        grid=(NPAD // blk,),
        in_specs=[
            pl.BlockSpec((blk, 128), lambda i: (i, 0)),
            pl.BlockSpec((blk, 2), lambda i: (i, 0)),
            pl.BlockSpec((1, 128), lambda i: (0, 0)),
            pl.BlockSpec((128, 128), lambda i: (0, 0)),
            pl.BlockSpec((128, 128), lambda i: (0, 0)),
        ],
        out_specs=[
            pl.BlockSpec((blk, 128), lambda i: (i, 0)),
            pl.BlockSpec((blk, 128), lambda i: (i, 0)),
        ],
        out_shape=[jax.ShapeDtypeStruct((NPAD, 128), jnp.float32)] * 2,
    )(acc, den, bias_flat, Wa, Wb)


def _mid2(acc, den, bias_flat, Wc_, al, ar):
    blk = 1024

    def body(a_ref, d_ref, b_ref, wc_ref, al_ref, ar_ref, h_ref, elr_ref):
        h = _normalize(a_ref, d_ref, b_ref)
        h_ref[...] = h
        f = jnp.dot(h, wc_ref[...], preferred_element_type=jnp.float32)
        el0 = jnp.sum(f[:, :256] * al_ref[0, :][None, :], axis=1, keepdims=True)
        el1 = jnp.sum(f[:, 256:] * al_ref[1, :][None, :], axis=1, keepdims=True)
        er0 = jnp.sum(f[:, :256] * ar_ref[0, :][None, :], axis=1, keepdims=True)
        er1 = jnp.sum(f[:, 256:] * ar_ref[1, :][None, :], axis=1, keepdims=True)
        elr_ref[...] = jnp.concatenate(
            [el0, el1, er0, er1, jnp.zeros((blk, 124), jnp.float32)], axis=1)

    return pl.pallas_call(
        body,
        grid=(NPAD // blk,),
        in_specs=[
            pl.BlockSpec((blk, 128), lambda i: (i, 0)),
            pl.BlockSpec((blk, 2), lambda i: (i, 0)),
            pl.BlockSpec((1, 128), lambda i: (0, 0)),
            pl.BlockSpec((128, 512), lambda i: (0, 0)),
            pl.BlockSpec((2, 256), lambda i: (0, 0)),
            pl.BlockSpec((2, 256), lambda i: (0, 0)),
        ],
        out_specs=[
            pl.BlockSpec((blk, 128), lambda i: (i, 0)),
            pl.BlockSpec((blk, 128), lambda i: (i, 0)),
        ],
        out_shape=[
            jax.ShapeDtypeStruct((NPAD, 128), jnp.float32),
            jax.ShapeDtypeStruct((NPAD, 128), jnp.float32),
        ],
    )(acc, den, bias_flat, Wc_, al, ar)


def _final(acc0, acc1, den3, Wc_, biasc_flat):
    blk = 1024

    def body(a0_ref, a1_ref, d_ref, wc_ref, b_ref, o_ref):
        g0 = a0_ref[...] / (d_ref[:, 0:1] + 1e-16)
        g1 = a1_ref[...] / (d_ref[:, 1:2] + 1e-16)
        o = (jnp.dot(g0, wc_ref[:, :256], preferred_element_type=jnp.float32)
             + jnp.dot(g1, wc_ref[:, 256:], preferred_element_type=jnp.float32)
             + b_ref[0, :256] + b_ref[0, 256:])
        o = o * 0.5
        m = jnp.max(o, axis=1, keepdims=True)
        ex = jnp.exp(o - m)
        o_ref[...] = ex / jnp.sum(ex, axis=1, keepdims=True)

    return pl.pallas_call(
        body,
        grid=(NPAD3 // blk,),
        in_specs=[
            pl.BlockSpec((blk, 128), lambda i: (i, 0)),
            pl.BlockSpec((blk, 128), lambda i: (i, 0)),
            pl.BlockSpec((blk, 2), lambda i: (i, 0)),
            pl.BlockSpec((128, 512), lambda i: (0, 0)),
            pl.BlockSpec((1, 512), lambda i: (0, 0)),
        ],
        out_specs=pl.BlockSpec((blk, 256), lambda i: (i, 0)),
        out_shape=jax.ShapeDtypeStruct((NPAD3, 256), jnp.float32),
    )(acc0, acc1, den3, Wc_, biasc_flat)


def kernel(in_feat, edge_index, emb, W_src1, W_dst1, attn1, bias1,
           W_src2, W_dst2, attn2, bias2, Wc, attn_lc, attn_rc, biasc):
    src = edge_index[0].astype(jnp.int32)
    dst = edge_index[1].astype(jnp.int32)
    srcp = jnp.concatenate([src, jnp.zeros((EPAD - E,), jnp.int32)])
    dstp = jnp.concatenate([dst, jnp.full((EPAD - E,), 1 << 28, jnp.int32)])
    inf3 = jnp.concatenate(
        [in_feat.astype(jnp.int32), jnp.zeros((NPAD - N,), jnp.int32)]
    ).reshape(NPAD // 256, 1, 256)

    fs1, fd1 = _prep(inf3, emb, W_src1, W_dst1)
    acc1, den1f = _edge_v2(fs1, fd1, srcp, dstp, attn1)
    den1 = den1f.reshape(NPAD, 2)
    fs2, fd2 = _mid(acc1, den1, bias1.reshape(1, 128), W_src2, W_dst2)
    acc2, den2f = _edge_v2(fs2, fd2, srcp, dstp, attn2)
    den2 = den2f.reshape(NPAD, 2)
    h2, elr = _mid2(acc2, den2, bias2.reshape(1, 128), Wc, attn_lc, attn_rc)
    acc30, acc31, den3f = _edge_gat(h2, elr, srcp, dstp)
    den3 = den3f.reshape(NPAD3, 2)
    out = _final(acc30, acc31, den3, Wc, biasc.reshape(1, 512))
    return out[:N]


# speedup vs baseline: 30.9547x; 1.0941x over previous
import functools

import jax
import jax.numpy as jnp
from jax import lax
from jax.experimental import pallas as pl
from jax.experimental.pallas import tpu as pltpu
from jax.experimental.pallas import tpu_sc as plsc

N = 50000
E = 800000
VOCAB = 256
EMB = 64

NR = 8192
RPC = 4
NPAD = NR * RPC * 2
NR3 = 3072
RPC3 = 9
NPAD3 = NR3 * RPC3 * 2
CB = 1024
NCHUNK = 49
EPAD = 16 * NCHUNK * CB
B = 128
HB = 1280
ZR = 16


def _i16(v):
    return jnp.full((16,), v, jnp.int32)


def _mo(x, m=8):
    return pl.multiple_of(x, m)


def _make_edge_v2():
    mesh = plsc.VectorSubcoreMesh(core_axis_name="c", subcore_axis_name="s")

    @functools.partial(
        pl.kernel,
        mesh=mesh,
        compiler_params=pltpu.CompilerParams(needs_layout_passes=False),
        out_type=[
            jax.ShapeDtypeStruct((NPAD, 128), jnp.float32),
            jax.ShapeDtypeStruct((NPAD * 2 // 128, 128), jnp.float32),
        ],
        scratch_types=[
            pltpu.VMEM_SHARED((NR + 8, 128), jnp.float32),
            pltpu.VMEM_SHARED((128, 128), jnp.float32),
            pltpu.VMEM((CB,), jnp.int32),
            pltpu.VMEM((CB,), jnp.int32),
            pltpu.VMEM((HB,), jnp.int32),
            pltpu.VMEM((HB,), jnp.int32),
            pltpu.VMEM((B,), jnp.int32),
            pltpu.VMEM((B,), jnp.int32),
            pltpu.VMEM((B,), jnp.int32),
            pltpu.VMEM((B + 16,), jnp.int32),
            pltpu.VMEM((B,), jnp.int32),
            pltpu.VMEM((B, 128), jnp.float32),
            pltpu.VMEM((B, 128), jnp.float32),
            pltpu.VMEM((136, 128), jnp.float32),
            pltpu.VMEM((ZR, 128), jnp.float32),
            pltpu.VMEM((2, 64), jnp.float32),
            pltpu.SemaphoreType.DMA,
            pltpu.SemaphoreType.DMA,
        ],
    )
    def k(fs_hbm, fd_hbm, src_hbm, dst_hbm, attn_hbm, acc_hbm, den_hbm,
          acc_sp, den_sp, sbuf, dbuf, hs, hd, gsrc, gdst, locb, locpad,
          ident, fsrow, fdrow, den_tile, zblk, attn_v, sem, sem2):
        c = lax.axis_index("c")
        s = lax.axis_index("s")
        iota16 = lax.iota(jnp.int32, 16)
        zeros16 = jnp.zeros((16,), jnp.float32)
        ebase = s * (EPAD // 16)
        lane0 = iota16 == 0

        pltpu.sync_copy(attn_hbm, attn_v)

        def zrow(r, _):
            for kk in range(8):
                zblk[r, pl.ds(kk * 16, 16)] = zeros16
            return 0
        lax.fori_loop(0, ZR, zrow, 0)
        for g in range(8):
            ident[pl.ds(g * 16, 16)] = iota16 + g * 16

        def process_block(off, valid, lo):
            def bg(g, _):
                dv = hd[pl.ds(off + g * 16, 16)]
                sv = hs[pl.ds(off + g * 16, 16)]
                m = (g * 16 + iota16) < valid
                lc = jnp.where(m, dv - lo, NR)
                gdst[pl.ds(g * 16, 16)] = jnp.where(m, dv, 0)
                gsrc[pl.ds(g * 16, 16)] = jnp.where(m, sv, 0)
                locb[pl.ds(g * 16, 16)] = lc
                locpad[pl.ds(g * 16, 16)] = lc
                return 0
            lax.fori_loop(0, B // 16, bg, 0)
            cp1 = pltpu.async_copy(fs_hbm.at[gsrc], fsrow, sem)
            cp2 = pltpu.async_copy(fd_hbm.at[gdst], fdrow, sem2)
            cp1.wait()
            cp2.wait()
            av = tuple(attn_v[hh, pl.ds(kk * 16, 16)]
                       for hh in range(2) for kk in range(4))

            def pedge(p, carry):
                x = [fsrow[p, pl.ds(kk * 16, 16)] for kk in range(8)]
                y = [fdrow[p, pl.ds(kk * 16, 16)] for kk in range(8)]
                w = []
                for kk in range(8):
                    t = x[kk] + y[kk]
                    w.append(jnp.maximum(t, 0.2 * t) * carry[kk])
                l0 = (w[0] + w[1]) + (w[2] + w[3])
                l1 = (w[4] + w[5]) + (w[6] + w[7])
                for sh in (1, 2, 4, 8):
                    l0 = l0 + l0[iota16 ^ sh]
                    l1 = l1 + l1[iota16 ^ sh]
                ex0 = jnp.exp(l0)
                ex1 = jnp.exp(l1)
                for kk in range(4):
                    fsrow[p, pl.ds(kk * 16, 16)] = x[kk] * ex0
                for kk in range(4, 8):
                    fsrow[p, pl.ds(kk * 16, 16)] = x[kk] * ex1
                ls = locpad[pl.ds(p, 16)][_i16(0)]
                dm = ls == NR
                d0 = jnp.where(dm, _i16(2 * NR), 2 * ls)
                d1 = jnp.where(dm, _i16(2 * NR), 2 * ls + 1)
                plsc.addupdate_scatter(
                    den_tile, [d0 >> 7, d0 & 127], ex0, mask=lane0)
                plsc.addupdate_scatter(
                    den_tile, [d1 >> 7, d1 & 127], ex1, mask=lane0)
                return carry
            lax.fori_loop(0, B, pedge, av)
            pltpu.sync_copy(fsrow, acc_sp.at[locb], add=True)

        def sweep(r, _):
            lo = (2 * r + c) * NR

            def zcp(j, _):
                row0 = _mo(s * (NR // 16) + j * ZR)
                pltpu.sync_copy(zblk, acc_sp.at[pl.ds(row0, ZR), :])
                return 0
            lax.fori_loop(0, NR // 16 // ZR, zcp, 0)

            def zdt(j, _):
                for kk in range(8):
                    den_tile[j, pl.ds(kk * 16, 16)] = zeros16
                return 0
            lax.fori_loop(0, 136, zdt, 0)
            pltpu.sync_copy(zblk.at[pl.ds(0, 8), :],
                            den_sp.at[pl.ds(_mo(s * 8), 8), :])
            plsc.subcore_barrier()

            def chunk(kk, cnt):
                pltpu.sync_copy(src_hbm.at[pl.ds(_mo(ebase + kk * CB), CB)], sbuf)
                pltpu.sync_copy(dst_hbm.at[pl.ds(_mo(ebase + kk * CB), CB)], dbuf)

                def scan(g, cnt):
                    dv = dbuf[pl.ds(g * 16, 16)]
                    sv = sbuf[pl.ds(g * 16, 16)]
                    m = (dv >= lo) & (dv < lo + NR)
                    pos = plsc.cumsum(m.astype(jnp.int32)) - 1 + cnt
                    plsc.store_scatter(hd, [pos], dv, mask=m)
                    plsc.store_scatter(hs, [pos], sv, mask=m)
                    return cnt + plsc.all_reduce_population_count(m)
                cnt = lax.fori_loop(0, CB // 16, scan, cnt)
                navail = cnt[0]
                nblk = navail // B

                def pb(i, _):
                    process_block(i * B, B, lo)
                    return 0
                lax.fori_loop(0, nblk, pb, 0)
                rem = navail - nblk * B

                def mv(g, _):
                    @pl.when(g * 16 < rem)
                    def _():
                        hd[pl.ds(g * 16, 16)] = hd[pl.ds(nblk * B + g * 16, 16)]
                        hs[pl.ds(g * 16, 16)] = hs[pl.ds(nblk * B + g * 16, 16)]
                    return 0
                lax.fori_loop(0, B // 16, mv, 0)
                return _i16(0) + rem
            cnt = lax.fori_loop(0, NCHUNK, chunk, _i16(0))
            rem = cnt[0]

            @pl.when(rem > 0)
            def _():
                process_block(0, rem, lo)
            plsc.subcore_barrier()
            pltpu.sync_copy(den_tile.at[pl.ds(0, 128), :],
                            den_sp.at[ident], add=True)
            plsc.subcore_barrier()

            def fcp(j, _):
                row0 = _mo(s * (NR // 16) + j * ZR)
                pltpu.sync_copy(acc_sp.at[pl.ds(row0, ZR), :],
                                acc_hbm.at[pl.ds(_mo(lo + row0), ZR), :])
                return 0
            lax.fori_loop(0, NR // 16 // ZR, fcp, 0)

            @pl.when(s == 0)
            def _():
                pltpu.sync_copy(den_sp,
                                den_hbm.at[pl.ds(_mo(lo // 64), 128), :])
            plsc.subcore_barrier()
            return 0
        lax.fori_loop(0, RPC, sweep, 0)

    return k


def _make_edge_gat():
    mesh = plsc.VectorSubcoreMesh(core_axis_name="c", subcore_axis_name="s")

    @functools.partial(
        pl.kernel,
        mesh=mesh,
        compiler_params=pltpu.CompilerParams(needs_layout_passes=False),
        out_type=[
            jax.ShapeDtypeStruct((NPAD3, 128), jnp.float32),
            jax.ShapeDtypeStruct((NPAD3, 128), jnp.float32),
            jax.ShapeDtypeStruct((NPAD3 * 2 // 128, 128), jnp.float32),
        ],
        scratch_types=[
            pltpu.VMEM_SHARED((NR3 + 8, 128), jnp.float32),
            pltpu.VMEM_SHARED((NR3 + 8, 128), jnp.float32),
            pltpu.VMEM_SHARED((128, 128), jnp.float32),
            pltpu.VMEM((CB,), jnp.int32),
            pltpu.VMEM((CB,), jnp.int32),
            pltpu.VMEM((HB,), jnp.int32),
            pltpu.VMEM((HB,), jnp.int32),
            pltpu.VMEM((B,), jnp.int32),
            pltpu.VMEM((B,), jnp.int32),
            pltpu.VMEM((B,), jnp.int32),
            pltpu.VMEM((B + 16,), jnp.int32),
            pltpu.VMEM((B,), jnp.int32),
            pltpu.VMEM((B + 16,), jnp.float32),
            pltpu.VMEM((B, 128), jnp.float32),
            pltpu.VMEM((B, 128), jnp.float32),
            pltpu.VMEM((B, 128), jnp.float32),
            pltpu.VMEM((136, 128), jnp.float32),
            pltpu.VMEM((ZR, 128), jnp.float32),
            pltpu.SemaphoreType.DMA,
            pltpu.SemaphoreType.DMA,
            pltpu.SemaphoreType.DMA,
        ],
    )
    def k(h2_hbm, elr_hbm, src_hbm, dst_hbm, acc0_hbm, acc1_hbm, den_hbm,
          acc0_sp, acc1_sp, den_sp, sbuf, dbuf, hs, hd, gsrc, gdst, locb,
          locpad, ident, exrb, h2row, elrow, errow, den_tile, zblk,
          sem, sem2, sem3):
        c = lax.axis_index("c")
        s = lax.axis_index("s")
        iota16 = lax.iota(jnp.int32, 16)
        zeros16 = jnp.zeros((16,), jnp.float32)
        ebase = s * (EPAD // 16)
        lane0 = iota16 == 0

        def zrow(r, _):
            for kk in range(8):
                zblk[r, pl.ds(kk * 16, 16)] = zeros16
            return 0
        lax.fori_loop(0, ZR, zrow, 0)
        for g in range(8):
            ident[pl.ds(g * 16, 16)] = iota16 + g * 16

        def process_block(off, valid, lo):
            def bg(g, _):
                dv = hd[pl.ds(off + g * 16, 16)]
                sv = hs[pl.ds(off + g * 16, 16)]
                m = (g * 16 + iota16) < valid
                lc = jnp.where(m, dv - lo, NR3)
                gdst[pl.ds(g * 16, 16)] = jnp.where(m, dv, 0)
                gsrc[pl.ds(g * 16, 16)] = jnp.where(m, sv, 0)
                locb[pl.ds(g * 16, 16)] = lc
                locpad[pl.ds(g * 16, 16)] = lc
                return 0
            lax.fori_loop(0, B // 16, bg, 0)
            cp1 = pltpu.async_copy(h2_hbm.at[gsrc], h2row, sem)
            cp2 = pltpu.async_copy(elr_hbm.at[gsrc], elrow, sem2)
            cp3 = pltpu.async_copy(elr_hbm.at[gdst], errow, sem3)
            cp1.wait()
            cp2.wait()
            cp3.wait()

            def pedge(p, _):
                evl = elrow[p, pl.ds(0, 16)]
                evr = errow[p, pl.ds(0, 16)]
                tv = evl + jnp.where(lane0, evr[_i16(2)], evr[_i16(3)])
                lv = jnp.maximum(tv, 0.2 * tv)
                ex0 = jnp.exp(lv[_i16(0)])
                ex1 = jnp.exp(lv[_i16(1)])
                for kk in range(8):
                    v = h2row[p, pl.ds(kk * 16, 16)]
                    h2row[p, pl.ds(kk * 16, 16)] = v * ex0
                plsc.store_scatter(exrb, [_i16(0) + p], ex1 / ex0, mask=lane0)
                ls = locpad[pl.ds(p, 16)][_i16(0)]
                dm = ls == NR3
                d0 = jnp.where(dm, _i16(2 * NR3), 2 * ls)
                d1 = jnp.where(dm, _i16(2 * NR3), 2 * ls + 1)
                plsc.addupdate_scatter(
                    den_tile, [d0 >> 7, d0 & 127], ex0, mask=lane0)
                plsc.addupdate_scatter(
                    den_tile, [d1 >> 7, d1 & 127], ex1, mask=lane0)
                return 0
            lax.fori_loop(0, B, pedge, 0)
            pltpu.sync_copy(h2row, acc0_sp.at[locb], add=True)

            def redge(p, _):
                rs = exrb[pl.ds(p, 16)][_i16(0)]
                for kk in range(8):
                    h2row[p, pl.ds(kk * 16, 16)] = (
                        h2row[p, pl.ds(kk * 16, 16)] * rs)
                return 0
            lax.fori_loop(0, B, redge, 0)
            pltpu.sync_copy(h2row, acc1_sp.at[locb], add=True)

        def sweep(r, _):
            lo = (2 * r + c) * NR3

            def zcp(j, _):
                row0 = _mo(s * (NR3 // 16) + j * ZR)
                pltpu.sync_copy(zblk, acc0_sp.at[pl.ds(row0, ZR), :])
                pltpu.sync_copy(zblk, acc1_sp.at[pl.ds(row0, ZR), :])
                return 0
            lax.fori_loop(0, NR3 // 16 // ZR, zcp, 0)

            def zdt(j, _):
                for kk in range(8):
                    den_tile[j, pl.ds(kk * 16, 16)] = zeros16
                return 0
            lax.fori_loop(0, 136, zdt, 0)
            pltpu.sync_copy(zblk.at[pl.ds(0, 8), :],
                            den_sp.at[pl.ds(_mo(s * 8), 8), :])
            plsc.subcore_barrier()

            def chunk(kk, cnt):
                pltpu.sync_copy(src_hbm.at[pl.ds(_mo(ebase + kk * CB), CB)], sbuf)
                pltpu.sync_copy(dst_hbm.at[pl.ds(_mo(ebase + kk * CB), CB)], dbuf)

                def scan(g, cnt):
                    dv = dbuf[pl.ds(g * 16, 16)]
                    sv = sbuf[pl.ds(g * 16, 16)]
                    m = (dv >= lo) & (dv < lo + NR3)
                    pos = plsc.cumsum(m.astype(jnp.int32)) - 1 + cnt
                    plsc.store_scatter(hd, [pos], dv, mask=m)
                    plsc.store_scatter(hs, [pos], sv, mask=m)
                    return cnt + plsc.all_reduce_population_count(m)
                cnt = lax.fori_loop(0, CB // 16, scan, cnt)
                navail = cnt[0]
                nblk = navail // B

                def pb(i, _):
                    process_block(i * B, B, lo)
                    return 0
                lax.fori_loop(0, nblk, pb, 0)
                rem = navail - nblk * B

                def mv(g, _):
                    @pl.when(g * 16 < rem)
                    def _():
                        hd[pl.ds(g * 16, 16)] = hd[pl.ds(nblk * B + g * 16, 16)]
                        hs[pl.ds(g * 16, 16)] = hs[pl.ds(nblk * B + g * 16, 16)]
                    return 0
                lax.fori_loop(0, B // 16, mv, 0)
                return _i16(0) + rem
            cnt = lax.fori_loop(0, NCHUNK, chunk, _i16(0))
            rem = cnt[0]

            @pl.when(rem > 0)
            def _():
                process_block(0, rem, lo)
            plsc.subcore_barrier()
            pltpu.sync_copy(den_tile.at[pl.ds(0, 128), :],
                            den_sp.at[ident], add=True)
            plsc.subcore_barrier()

            def fcp(j, _):
                row0 = _mo(s * (NR3 // 16) + j * ZR)
                pltpu.sync_copy(acc0_sp.at[pl.ds(row0, ZR), :],
                                acc0_hbm.at[pl.ds(_mo(lo + row0), ZR), :])
                pltpu.sync_copy(acc1_sp.at[pl.ds(row0, ZR), :],
                                acc1_hbm.at[pl.ds(_mo(lo + row0), ZR), :])
                return 0
            lax.fori_loop(0, NR3 // 16 // ZR, fcp, 0)

            @pl.when(s == 0)
            def _():
                pltpu.sync_copy(den_sp.at[pl.ds(0, 48), :],
                                den_hbm.at[pl.ds(_mo(lo // 64), 48), :])
            plsc.subcore_barrier()
            return 0
        lax.fori_loop(0, RPC3, sweep, 0)

    return k


_edge_v2 = _make_edge_v2()
_edge_gat = _make_edge_gat()


def _prep(in_feat3, emb, Ws, Wd):
    def body(if_ref, emb_ref, ws_ref, wd_ref, fs_ref, fd_ref):
        ids = if_ref[0, 0, :]
        oh = (ids[:, None] == lax.broadcasted_iota(jnp.int32, (256, VOCAB), 1)
              ).astype(jnp.float32)
        h = jnp.dot(oh, emb_ref[...], preferred_element_type=jnp.float32)
        fs_ref[...] = jnp.dot(h, ws_ref[...], preferred_element_type=jnp.float32)
        fd_ref[...] = jnp.dot(h, wd_ref[...], preferred_element_type=jnp.float32)

    return pl.pallas_call(
        body,
        grid=(NPAD // 256,),
        in_specs=[
            pl.BlockSpec((1, 1, 256), lambda i: (i, 0, 0)),
            pl.BlockSpec((VOCAB, EMB), lambda i: (0, 0)),
            pl.BlockSpec((EMB, 128), lambda i: (0, 0)),
            pl.BlockSpec((EMB, 128), lambda i: (0, 0)),
        ],
        out_specs=[
            pl.BlockSpec((256, 128), lambda i: (i, 0)),
            pl.BlockSpec((256, 128), lambda i: (i, 0)),
        ],
        out_shape=[jax.ShapeDtypeStruct((NPAD, 128), jnp.float32)] * 2,
    )(in_feat3, emb, Ws, Wd)


def _normalize(a_ref, d_ref, b_ref):
    den0 = d_ref[:, 0:1] + 1e-16
    den1 = d_ref[:, 1:2] + 1e-16
    h = jnp.concatenate([a_ref[:, :64] / den0, a_ref[:, 64:] / den1], axis=1)
    h = h + b_ref[...]
    return jnp.where(h > 0, h, jnp.exp(jnp.minimum(h, 0.0)) - 1.0)


def _mid(acc, den, bias_flat, Wa, Wb):
    blk = 1024

    def body(a_ref, d_ref, b_ref, wa_ref, wb_ref, fs_ref, fd_ref):
        h = _normalize(a_ref, d_ref, b_ref)
        fs_ref[...] = jnp.dot(h, wa_ref[...], preferred_element_type=jnp.float32)
        fd_ref[...] = jnp.dot(h, wb_ref[...], preferred_element_type=jnp.float32)

    return pl.pallas_call(
        body,
        grid=(NPAD // blk,),
        in_specs=[
            pl.BlockSpec((blk, 128), lambda i: (i, 0)),
            pl.BlockSpec((blk, 2), lambda i: (i, 0)),
            pl.BlockSpec((1, 128), lambda i: (0, 0)),
            pl.BlockSpec((128, 128), lambda i: (0, 0)),
            pl.BlockSpec((128, 128), lambda i: (0, 0)),
        ],
        out_specs=[
            pl.BlockSpec((blk, 128), lambda i: (i, 0)),
            pl.BlockSpec((blk, 128), lambda i: (i, 0)),
        ],
        out_shape=[jax.ShapeDtypeStruct((NPAD, 128), jnp.float32)] * 2,
    )(acc, den, bias_flat, Wa, Wb)


def _mid2(acc, den, bias_flat, Wc_, al, ar):
    blk = 1024

    def body(a_ref, d_ref, b_ref, wc_ref, al_ref, ar_ref, h_ref, elr_ref):
        h = _normalize(a_ref, d_ref, b_ref)
        h_ref[...] = h
        f = jnp.dot(h, wc_ref[...], preferred_element_type=jnp.float32)
        el0 = jnp.sum(f[:, :256] * al_ref[0, :][None, :], axis=1, keepdims=True)
        el1 = jnp.sum(f[:, 256:] * al_ref[1, :][None, :], axis=1, keepdims=True)
        er0 = jnp.sum(f[:, :256] * ar_ref[0, :][None, :], axis=1, keepdims=True)
        er1 = jnp.sum(f[:, 256:] * ar_ref[1, :][None, :], axis=1, keepdims=True)
        elr_ref[...] = jnp.concatenate(
            [el0, el1, er0, er1, jnp.zeros((blk, 124), jnp.float32)], axis=1)

    return pl.pallas_call(
        body,
        grid=(NPAD // blk,),
        in_specs=[
            pl.BlockSpec((blk, 128), lambda i: (i, 0)),
            pl.BlockSpec((blk, 2), lambda i: (i, 0)),
            pl.BlockSpec((1, 128), lambda i: (0, 0)),
            pl.BlockSpec((128, 512), lambda i: (0, 0)),
            pl.BlockSpec((2, 256), lambda i: (0, 0)),
            pl.BlockSpec((2, 256), lambda i: (0, 0)),
        ],
        out_specs=[
            pl.BlockSpec((blk, 128), lambda i: (i, 0)),
            pl.BlockSpec((blk, 128), lambda i: (i, 0)),
        ],
        out_shape=[
            jax.ShapeDtypeStruct((NPAD, 128), jnp.float32),
            jax.ShapeDtypeStruct((NPAD, 128), jnp.float32),
        ],
    )(acc, den, bias_flat, Wc_, al, ar)


def _final(acc0, acc1, den3, Wc_, biasc_flat):
    blk = 1024

    def body(a0_ref, a1_ref, d_ref, wc_ref, b_ref, o_ref):
        g0 = a0_ref[...] / (d_ref[:, 0:1] + 1e-16)
        g1 = a1_ref[...] / (d_ref[:, 1:2] + 1e-16)
        o = (jnp.dot(g0, wc_ref[:, :256], preferred_element_type=jnp.float32)
             + jnp.dot(g1, wc_ref[:, 256:], preferred_element_type=jnp.float32)
             + b_ref[0, :256] + b_ref[0, 256:])
        o = o * 0.5
        m = jnp.max(o, axis=1, keepdims=True)
        ex = jnp.exp(o - m)
        o_ref[...] = ex / jnp.sum(ex, axis=1, keepdims=True)

    return pl.pallas_call(
        body,
        grid=(NPAD3 // blk,),
        in_specs=[
            pl.BlockSpec((blk, 128), lambda i: (i, 0)),
            pl.BlockSpec((blk, 128), lambda i: (i, 0)),
            pl.BlockSpec((blk, 2), lambda i: (i, 0)),
            pl.BlockSpec((128, 512), lambda i: (0, 0)),
            pl.BlockSpec((1, 512), lambda i: (0, 0)),
        ],
        out_specs=pl.BlockSpec((blk, 256), lambda i: (i, 0)),
        out_shape=jax.ShapeDtypeStruct((NPAD3, 256), jnp.float32),
    )(acc0, acc1, den3, Wc_, biasc_flat)


def kernel(in_feat, edge_index, emb, W_src1, W_dst1, attn1, bias1,
           W_src2, W_dst2, attn2, bias2, Wc, attn_lc, attn_rc, biasc):
    src = edge_index[0].astype(jnp.int32)
    dst = edge_index[1].astype(jnp.int32)
    srcp = jnp.concatenate([src, jnp.zeros((EPAD - E,), jnp.int32)])
    dstp = jnp.concatenate([dst, jnp.full((EPAD - E,), 1 << 28, jnp.int32)])
    inf3 = jnp.concatenate(
        [in_feat.astype(jnp.int32), jnp.zeros((NPAD - N,), jnp.int32)]
    ).reshape(NPAD // 256, 1, 256)

    fs1, fd1 = _prep(inf3, emb, W_src1, W_dst1)
    acc1, den1f = _edge_v2(fs1, fd1, srcp, dstp, attn1)
    den1 = den1f.reshape(NPAD, 2)
    fs2, fd2 = _mid(acc1, den1, bias1.reshape(1, 128), W_src2, W_dst2)
    acc2, den2f = _edge_v2(fs2, fd2, srcp, dstp, attn2)
    den2 = den2f.reshape(NPAD, 2)
    h2, elr = _mid2(acc2, den2, bias2.reshape(1, 128), Wc, attn_lc, attn_rc)
    acc30, acc31, den3f = _edge_gat(h2, elr, srcp, dstp)
    den3 = den3f.reshape(NPAD3, 2)
    out = _final(acc30, acc31, den3, Wc, biasc.reshape(1, 512))
    return out[:N]


# speedup vs baseline: 35.2671x; 1.1393x over previous
import functools

import jax
import jax.numpy as jnp
from jax import lax
from jax.experimental import pallas as pl
from jax.experimental.pallas import tpu as pltpu
from jax.experimental.pallas import tpu_sc as plsc

N = 50000
E = 800000
VOCAB = 256
EMB = 64

NR = 8192
RPC = 4
NPAD = NR * RPC * 2
NR3 = 3584
RPC3 = 7
NPAD3 = NR3 * RPC3 * 2
CB = 1024
NCHUNK = 49
EPAD = 16 * NCHUNK * CB
B = 128
HB = 1280
ZR = 16


def _i16(v):
    return jnp.full((16,), v, jnp.int32)


def _mo(x, m=8):
    return pl.multiple_of(x, m)


def _make_edge_v2():
    mesh = plsc.VectorSubcoreMesh(core_axis_name="c", subcore_axis_name="s")

    @functools.partial(
        pl.kernel,
        mesh=mesh,
        compiler_params=pltpu.CompilerParams(needs_layout_passes=False),
        out_type=[
            jax.ShapeDtypeStruct((NPAD, 128), jnp.float32),
            jax.ShapeDtypeStruct((NPAD * 2 // 128, 128), jnp.float32),
        ],
        scratch_types=[
            pltpu.VMEM_SHARED((NR + 8, 128), jnp.float32),
            pltpu.VMEM_SHARED((128, 128), jnp.float32),
            pltpu.VMEM((CB,), jnp.int32),
            pltpu.VMEM((CB,), jnp.int32),
            pltpu.VMEM((HB,), jnp.int32),
            pltpu.VMEM((HB,), jnp.int32),
            pltpu.VMEM((B,), jnp.int32),
            pltpu.VMEM((B,), jnp.int32),
            pltpu.VMEM((B,), jnp.int32),
            pltpu.VMEM((B + 16,), jnp.int32),
            pltpu.VMEM((B,), jnp.int32),
            pltpu.VMEM((B, 128), jnp.float32),
            pltpu.VMEM((B, 128), jnp.float32),
            pltpu.VMEM((136, 128), jnp.float32),
            pltpu.VMEM((ZR, 128), jnp.float32),
            pltpu.VMEM((2, 64), jnp.float32),
            pltpu.SemaphoreType.DMA,
            pltpu.SemaphoreType.DMA,
        ],
    )
    def k(fs_hbm, fd_hbm, src_hbm, dst_hbm, attn_hbm, acc_hbm, den_hbm,
          acc_sp, den_sp, sbuf, dbuf, hs, hd, gsrc, gdst, locb, locpad,
          ident, fsrow, fdrow, den_tile, zblk, attn_v, sem, sem2):
        c = lax.axis_index("c")
        s = lax.axis_index("s")
        iota16 = lax.iota(jnp.int32, 16)
        zeros16 = jnp.zeros((16,), jnp.float32)
        ebase = s * (EPAD // 16)
        lane0 = iota16 == 0

        pltpu.sync_copy(attn_hbm, attn_v)

        def zrow(r, _):
            for kk in range(8):
                zblk[r, pl.ds(kk * 16, 16)] = zeros16
            return 0
        lax.fori_loop(0, ZR, zrow, 0)
        for g in range(8):
            ident[pl.ds(g * 16, 16)] = iota16 + g * 16

        def process_block(off, valid, lo):
            def bg(g, _):
                dv = hd[pl.ds(off + g * 16, 16)]
                sv = hs[pl.ds(off + g * 16, 16)]
                m = (g * 16 + iota16) < valid
                lc = jnp.where(m, dv - lo, NR)
                gdst[pl.ds(g * 16, 16)] = jnp.where(m, dv, 0)
                gsrc[pl.ds(g * 16, 16)] = jnp.where(m, sv, 0)
                locb[pl.ds(g * 16, 16)] = lc
                locpad[pl.ds(g * 16, 16)] = lc
                return 0
            lax.fori_loop(0, B // 16, bg, 0)
            cp1 = pltpu.async_copy(fs_hbm.at[gsrc], fsrow, sem)
            cp2 = pltpu.async_copy(fd_hbm.at[gdst], fdrow, sem2)
            cp1.wait()
            cp2.wait()
            av = tuple(attn_v[hh, pl.ds(kk * 16, 16)]
                       for hh in range(2) for kk in range(4))

            def pedge(p, carry):
                x = [fsrow[p, pl.ds(kk * 16, 16)] for kk in range(8)]
                y = [fdrow[p, pl.ds(kk * 16, 16)] for kk in range(8)]
                w = []
                for kk in range(8):
                    t = x[kk] + y[kk]
                    w.append(jnp.maximum(t, 0.2 * t) * carry[kk])
                l0 = (w[0] + w[1]) + (w[2] + w[3])
                l1 = (w[4] + w[5]) + (w[6] + w[7])
                for sh in (1, 2, 4, 8):
                    l0 = l0 + l0[iota16 ^ sh]
                    l1 = l1 + l1[iota16 ^ sh]
                ex0 = jnp.exp(l0)
                ex1 = jnp.exp(l1)
                for kk in range(4):
                    fsrow[p, pl.ds(kk * 16, 16)] = x[kk] * ex0
                for kk in range(4, 8):
                    fsrow[p, pl.ds(kk * 16, 16)] = x[kk] * ex1
                ls = locpad[pl.ds(p, 16)][_i16(0)]
                dm = ls == NR
                d0 = jnp.where(dm, _i16(2 * NR), 2 * ls)
                d1 = jnp.where(dm, _i16(2 * NR), 2 * ls + 1)
                plsc.addupdate_scatter(
                    den_tile, [d0 >> 7, d0 & 127], ex0, mask=lane0)
                plsc.addupdate_scatter(
                    den_tile, [d1 >> 7, d1 & 127], ex1, mask=lane0)
                return carry
            lax.fori_loop(0, B, pedge, av)
            pltpu.sync_copy(fsrow, acc_sp.at[locb], add=True)

        def sweep(r, _):
            lo = (2 * r + c) * NR

            def zcp(j, _):
                row0 = _mo(s * (NR // 16) + j * ZR)
                pltpu.sync_copy(zblk, acc_sp.at[pl.ds(row0, ZR), :])
                return 0
            lax.fori_loop(0, NR // 16 // ZR, zcp, 0)

            def zdt(j, _):
                for kk in range(8):
                    den_tile[j, pl.ds(kk * 16, 16)] = zeros16
                return 0
            lax.fori_loop(0, 136, zdt, 0)
            pltpu.sync_copy(zblk.at[pl.ds(0, 8), :],
                            den_sp.at[pl.ds(_mo(s * 8), 8), :])
            plsc.subcore_barrier()

            def chunk(kk, cnt):
                pltpu.sync_copy(src_hbm.at[pl.ds(_mo(ebase + kk * CB), CB)], sbuf)
                pltpu.sync_copy(dst_hbm.at[pl.ds(_mo(ebase + kk * CB), CB)], dbuf)

                def scan(g, cnt):
                    dv = dbuf[pl.ds(g * 16, 16)]
                    sv = sbuf[pl.ds(g * 16, 16)]
                    m = (dv >= lo) & (dv < lo + NR)
                    pos = plsc.cumsum(m.astype(jnp.int32)) - 1 + cnt
                    plsc.store_scatter(hd, [pos], dv, mask=m)
                    plsc.store_scatter(hs, [pos], sv, mask=m)
                    return cnt + plsc.all_reduce_population_count(m)
                cnt = lax.fori_loop(0, CB // 16, scan, cnt)
                navail = cnt[0]
                nblk = navail // B

                def pb(i, _):
                    process_block(i * B, B, lo)
                    return 0
                lax.fori_loop(0, nblk, pb, 0)
                rem = navail - nblk * B

                def mv(g, _):
                    @pl.when(g * 16 < rem)
                    def _():
                        hd[pl.ds(g * 16, 16)] = hd[pl.ds(nblk * B + g * 16, 16)]
                        hs[pl.ds(g * 16, 16)] = hs[pl.ds(nblk * B + g * 16, 16)]
                    return 0
                lax.fori_loop(0, B // 16, mv, 0)
                return _i16(0) + rem
            cnt = lax.fori_loop(0, NCHUNK, chunk, _i16(0))
            rem = cnt[0]

            @pl.when(rem > 0)
            def _():
                process_block(0, rem, lo)
            plsc.subcore_barrier()
            pltpu.sync_copy(den_tile.at[pl.ds(0, 128), :],
                            den_sp.at[ident], add=True)
            plsc.subcore_barrier()

            def fcp(j, _):
                row0 = _mo(s * (NR // 16) + j * ZR)
                pltpu.sync_copy(acc_sp.at[pl.ds(row0, ZR), :],
                                acc_hbm.at[pl.ds(_mo(lo + row0), ZR), :])
                return 0
            lax.fori_loop(0, NR // 16 // ZR, fcp, 0)

            @pl.when(s == 0)
            def _():
                pltpu.sync_copy(den_sp,
                                den_hbm.at[pl.ds(_mo(lo // 64), 128), :])
            plsc.subcore_barrier()
            return 0
        lax.fori_loop(0, RPC, sweep, 0)

    return k


def _make_edge_gat():
    mesh = plsc.VectorSubcoreMesh(core_axis_name="c", subcore_axis_name="s")

    @functools.partial(
        pl.kernel,
        mesh=mesh,
        compiler_params=pltpu.CompilerParams(needs_layout_passes=False),
        out_type=[
            jax.ShapeDtypeStruct((NPAD3, 128), jnp.float32),
            jax.ShapeDtypeStruct((NPAD3, 128), jnp.float32),
            jax.ShapeDtypeStruct((NPAD3 * 2 // 128, 128), jnp.float32),
        ],
        scratch_types=[
            pltpu.VMEM_SHARED((NR3 + 8, 128), jnp.float32),
            pltpu.VMEM_SHARED((NR3 + 8, 128), jnp.float32),
            pltpu.VMEM_SHARED((128, 128), jnp.float32),
            pltpu.VMEM((CB,), jnp.int32),
            pltpu.VMEM((CB,), jnp.int32),
            pltpu.VMEM((HB,), jnp.int32),
            pltpu.VMEM((HB,), jnp.int32),
            pltpu.VMEM((B,), jnp.int32),
            pltpu.VMEM((B,), jnp.int32),
            pltpu.VMEM((B,), jnp.int32),
            pltpu.VMEM((B + 16,), jnp.int32),
            pltpu.VMEM((B,), jnp.int32),
            pltpu.VMEM((B + 16,), jnp.float32),
            pltpu.VMEM((B, 128), jnp.float32),
            pltpu.VMEM((B, 128), jnp.float32),
            pltpu.VMEM((NR3 + 16,), jnp.float32),
            pltpu.VMEM((NR3 + 16,), jnp.float32),
            pltpu.VMEM((136, 128), jnp.float32),
            pltpu.VMEM((ZR, 128), jnp.float32),
            pltpu.SemaphoreType.DMA,
            pltpu.SemaphoreType.DMA,
        ],
    )
    def k(h2_hbm, elr_hbm, erp_hbm, src_hbm, dst_hbm, acc0_hbm, acc1_hbm,
          den_hbm, acc0_sp, acc1_sp, den_sp, sbuf, dbuf, hs, hd, gsrc, gdst,
          locb, locpad, ident, exrb, h2row, elrow, er0rng, er1rng, den_tile,
          zblk, sem, sem2):
        c = lax.axis_index("c")
        s = lax.axis_index("s")
        iota16 = lax.iota(jnp.int32, 16)
        zeros16 = jnp.zeros((16,), jnp.float32)
        ebase = s * (EPAD // 16)
        lane0 = iota16 == 0

        def zrow(r, _):
            for kk in range(8):
                zblk[r, pl.ds(kk * 16, 16)] = zeros16
            return 0
        lax.fori_loop(0, ZR, zrow, 0)
        for g in range(8):
            ident[pl.ds(g * 16, 16)] = iota16 + g * 16

        def process_block(off, valid, lo):
            def bg(g, _):
                dv = hd[pl.ds(off + g * 16, 16)]
                sv = hs[pl.ds(off + g * 16, 16)]
                m = (g * 16 + iota16) < valid
                lc = jnp.where(m, dv - lo, NR3)
                gdst[pl.ds(g * 16, 16)] = jnp.where(m, dv, 0)
                gsrc[pl.ds(g * 16, 16)] = jnp.where(m, sv, 0)
                locb[pl.ds(g * 16, 16)] = lc
                locpad[pl.ds(g * 16, 16)] = lc
                return 0
            lax.fori_loop(0, B // 16, bg, 0)
            cp1 = pltpu.async_copy(h2_hbm.at[gsrc], h2row, sem)
            cp2 = pltpu.async_copy(elr_hbm.at[gsrc], elrow, sem2)
            cp1.wait()
            cp2.wait()

            def pedge(p, _):
                evl = elrow[p, pl.ds(0, 16)]
                ls = locpad[pl.ds(p, 16)][_i16(0)]
                t0 = evl[_i16(0)] + plsc.load_gather(er0rng, [ls])
                t1 = evl[_i16(1)] + plsc.load_gather(er1rng, [ls])
                ex0 = jnp.exp(jnp.maximum(t0, 0.2 * t0))
                ex1 = jnp.exp(jnp.maximum(t1, 0.2 * t1))
                for kk in range(8):
                    v = h2row[p, pl.ds(kk * 16, 16)]
                    h2row[p, pl.ds(kk * 16, 16)] = v * ex0
                plsc.store_scatter(exrb, [_i16(0) + p], ex1 / ex0, mask=lane0)
                dm = ls == NR3
                d0 = jnp.where(dm, _i16(2 * NR3), 2 * ls)
                d1 = jnp.where(dm, _i16(2 * NR3), 2 * ls + 1)
                plsc.addupdate_scatter(
                    den_tile, [d0 >> 7, d0 & 127], ex0, mask=lane0)
                plsc.addupdate_scatter(
                    den_tile, [d1 >> 7, d1 & 127], ex1, mask=lane0)
                return 0
            lax.fori_loop(0, B, pedge, 0)
            pltpu.sync_copy(h2row, acc0_sp.at[locb], add=True)

            def redge(p, _):
                rs = exrb[pl.ds(p, 16)][_i16(0)]
                for kk in range(8):
                    h2row[p, pl.ds(kk * 16, 16)] = (
                        h2row[p, pl.ds(kk * 16, 16)] * rs)
                return 0
            lax.fori_loop(0, B, redge, 0)
            pltpu.sync_copy(h2row, acc1_sp.at[locb], add=True)

        def sweep(r, _):
            lo = (2 * r + c) * NR3
            pltpu.sync_copy(erp_hbm.at[0, pl.ds(lo, NR3)],
                            er0rng.at[pl.ds(0, NR3)])
            pltpu.sync_copy(erp_hbm.at[1, pl.ds(lo, NR3)],
                            er1rng.at[pl.ds(0, NR3)])

            def zcp(j, _):
                row0 = _mo(s * (NR3 // 16) + j * ZR)
                pltpu.sync_copy(zblk, acc0_sp.at[pl.ds(row0, ZR), :])
                pltpu.sync_copy(zblk, acc1_sp.at[pl.ds(row0, ZR), :])
                return 0
            lax.fori_loop(0, NR3 // 16 // ZR, zcp, 0)

            def zdt(j, _):
                for kk in range(8):
                    den_tile[j, pl.ds(kk * 16, 16)] = zeros16
                return 0
            lax.fori_loop(0, 136, zdt, 0)
            pltpu.sync_copy(zblk.at[pl.ds(0, 8), :],
                            den_sp.at[pl.ds(_mo(s * 8), 8), :])
            plsc.subcore_barrier()

            def chunk(kk, cnt):
                pltpu.sync_copy(src_hbm.at[pl.ds(_mo(ebase + kk * CB), CB)], sbuf)
                pltpu.sync_copy(dst_hbm.at[pl.ds(_mo(ebase + kk * CB), CB)], dbuf)

                def scan(g, cnt):
                    dv = dbuf[pl.ds(g * 16, 16)]
                    sv = sbuf[pl.ds(g * 16, 16)]
                    m = (dv >= lo) & (dv < lo + NR3)
                    pos = plsc.cumsum(m.astype(jnp.int32)) - 1 + cnt
                    plsc.store_scatter(hd, [pos], dv, mask=m)
                    plsc.store_scatter(hs, [pos], sv, mask=m)
                    return cnt + plsc.all_reduce_population_count(m)
                cnt = lax.fori_loop(0, CB // 16, scan, cnt)
                navail = cnt[0]
                nblk = navail // B

                def pb(i, _):
                    process_block(i * B, B, lo)
                    return 0
                lax.fori_loop(0, nblk, pb, 0)
                rem = navail - nblk * B

                def mv(g, _):
                    @pl.when(g * 16 < rem)
                    def _():
                        hd[pl.ds(g * 16, 16)] = hd[pl.ds(nblk * B + g * 16, 16)]
                        hs[pl.ds(g * 16, 16)] = hs[pl.ds(nblk * B + g * 16, 16)]
                    return 0
                lax.fori_loop(0, B // 16, mv, 0)
                return _i16(0) + rem
            cnt = lax.fori_loop(0, NCHUNK, chunk, _i16(0))
            rem = cnt[0]

            @pl.when(rem > 0)
            def _():
                process_block(0, rem, lo)
            plsc.subcore_barrier()
            pltpu.sync_copy(den_tile.at[pl.ds(0, 128), :],
                            den_sp.at[ident], add=True)
            plsc.subcore_barrier()

            def fcp(j, _):
                row0 = _mo(s * (NR3 // 16) + j * ZR)
                pltpu.sync_copy(acc0_sp.at[pl.ds(row0, ZR), :],
                                acc0_hbm.at[pl.ds(_mo(lo + row0), ZR), :])
                pltpu.sync_copy(acc1_sp.at[pl.ds(row0, ZR), :],
                                acc1_hbm.at[pl.ds(_mo(lo + row0), ZR), :])
                return 0
            lax.fori_loop(0, NR3 // 16 // ZR, fcp, 0)

            @pl.when(s == 0)
            def _():
                pltpu.sync_copy(den_sp.at[pl.ds(0, 56), :],
                                den_hbm.at[pl.ds(_mo(lo // 64), 56), :])
            plsc.subcore_barrier()
            return 0
        lax.fori_loop(0, RPC3, sweep, 0)

    return k


_edge_v2 = _make_edge_v2()
_edge_gat = _make_edge_gat()


def _prep(in_feat3, emb, Ws, Wd):
    def body(if_ref, emb_ref, ws_ref, wd_ref, fs_ref, fd_ref):
        ids = if_ref[0, 0, :]
        oh = (ids[:, None] == lax.broadcasted_iota(jnp.int32, (256, VOCAB), 1)
              ).astype(jnp.float32)
        h = jnp.dot(oh, emb_ref[...], preferred_element_type=jnp.float32)
        fs_ref[...] = jnp.dot(h, ws_ref[...], preferred_element_type=jnp.float32)
        fd_ref[...] = jnp.dot(h, wd_ref[...], preferred_element_type=jnp.float32)

    return pl.pallas_call(
        body,
        grid=(NPAD // 256,),
        in_specs=[
            pl.BlockSpec((1, 1, 256), lambda i: (i, 0, 0)),
            pl.BlockSpec((VOCAB, EMB), lambda i: (0, 0)),
            pl.BlockSpec((EMB, 128), lambda i: (0, 0)),
            pl.BlockSpec((EMB, 128), lambda i: (0, 0)),
        ],
        out_specs=[
            pl.BlockSpec((256, 128), lambda i: (i, 0)),
            pl.BlockSpec((256, 128), lambda i: (i, 0)),
        ],
        out_shape=[jax.ShapeDtypeStruct((NPAD, 128), jnp.float32)] * 2,
    )(in_feat3, emb, Ws, Wd)


def _normalize(a_ref, d_ref, b_ref):
    den0 = d_ref[:, 0:1] + 1e-16
    den1 = d_ref[:, 1:2] + 1e-16
    h = jnp.concatenate([a_ref[:, :64] / den0, a_ref[:, 64:] / den1], axis=1)
    h = h + b_ref[...]
    return jnp.where(h > 0, h, jnp.exp(jnp.minimum(h, 0.0)) - 1.0)


def _mid(acc, den, bias_flat, Wa, Wb):
    blk = 1024

    def body(a_ref, d_ref, b_ref, wa_ref, wb_ref, fs_ref, fd_ref):
        h = _normalize(a_ref, d_ref, b_ref)
        fs_ref[...] = jnp.dot(h, wa_ref[...], preferred_element_type=jnp.float32)
        fd_ref[...] = jnp.dot(h, wb_ref[...], preferred_element_type=jnp.float32)

    return pl.pallas_call(
        body,
        grid=(NPAD // blk,),
        in_specs=[
            pl.BlockSpec((blk, 128), lambda i: (i, 0)),
            pl.BlockSpec((blk, 2), lambda i: (i, 0)),
            pl.BlockSpec((1, 128), lambda i: (0, 0)),
            pl.BlockSpec((128, 128), lambda i: (0, 0)),
            pl.BlockSpec((128, 128), lambda i: (0, 0)),
        ],
        out_specs=[
            pl.BlockSpec((blk, 128), lambda i: (i, 0)),
            pl.BlockSpec((blk, 128), lambda i: (i, 0)),
        ],
        out_shape=[jax.ShapeDtypeStruct((NPAD, 128), jnp.float32)] * 2,
    )(acc, den, bias_flat, Wa, Wb)


def _mid2(acc, den, bias_flat, Wc_, al, ar):
    blk = 1024

    def body(a_ref, d_ref, b_ref, wc_ref, al_ref, ar_ref, h_ref, elr_ref,
             erp_ref):
        h = _normalize(a_ref, d_ref, b_ref)
        h_ref[...] = h
        f = jnp.dot(h, wc_ref[...], preferred_element_type=jnp.float32)
        el0 = jnp.sum(f[:, :256] * al_ref[0, :][None, :], axis=1, keepdims=True)
        el1 = jnp.sum(f[:, 256:] * al_ref[1, :][None, :], axis=1, keepdims=True)
        elr_ref[...] = jnp.concatenate(
            [el0, el1, jnp.zeros((blk, 126), jnp.float32)], axis=1)
        dn = (((1,), (1,)), ((), ()))
        wr0 = lax.dot_general(ar_ref[0:1, :], wc_ref[:, :256], dn,
                              preferred_element_type=jnp.float32)
        wr1 = lax.dot_general(ar_ref[1:2, :], wc_ref[:, 256:], dn,
                              preferred_element_type=jnp.float32)
        er0 = lax.dot_general(wr0, h, dn, preferred_element_type=jnp.float32)
        er1 = lax.dot_general(wr1, h, dn, preferred_element_type=jnp.float32)
        erp_ref[...] = jnp.concatenate([er0, er1], axis=0)

    return pl.pallas_call(
        body,
        grid=(NPAD // blk,),
        in_specs=[
            pl.BlockSpec((blk, 128), lambda i: (i, 0)),
            pl.BlockSpec((blk, 2), lambda i: (i, 0)),
            pl.BlockSpec((1, 128), lambda i: (0, 0)),
            pl.BlockSpec((128, 512), lambda i: (0, 0)),
            pl.BlockSpec((2, 256), lambda i: (0, 0)),
            pl.BlockSpec((2, 256), lambda i: (0, 0)),
        ],
        out_specs=[
            pl.BlockSpec((blk, 128), lambda i: (i, 0)),
            pl.BlockSpec((blk, 128), lambda i: (i, 0)),
            pl.BlockSpec((2, blk), lambda i: (0, i)),
        ],
        out_shape=[
            jax.ShapeDtypeStruct((NPAD, 128), jnp.float32),
            jax.ShapeDtypeStruct((NPAD, 128), jnp.float32),
            jax.ShapeDtypeStruct((2, NPAD), jnp.float32),
        ],
    )(acc, den, bias_flat, Wc_, al, ar)


def _final(acc0, acc1, den3, Wc_, biasc_flat):
    blk = 1024

    def body(a0_ref, a1_ref, d_ref, wc_ref, b_ref, o_ref):
        g0 = a0_ref[...] / (d_ref[:, 0:1] + 1e-16)
        g1 = a1_ref[...] / (d_ref[:, 1:2] + 1e-16)
        o = (jnp.dot(g0, wc_ref[:, :256], preferred_element_type=jnp.float32)
             + jnp.dot(g1, wc_ref[:, 256:], preferred_element_type=jnp.float32)
             + b_ref[0, :256] + b_ref[0, 256:])
        o = o * 0.5
        m = jnp.max(o, axis=1, keepdims=True)
        ex = jnp.exp(o - m)
        o_ref[...] = ex / jnp.sum(ex, axis=1, keepdims=True)

    return pl.pallas_call(
        body,
        grid=(NPAD3 // blk,),
        in_specs=[
            pl.BlockSpec((blk, 128), lambda i: (i, 0)),
            pl.BlockSpec((blk, 128), lambda i: (i, 0)),
            pl.BlockSpec((blk, 2), lambda i: (i, 0)),
            pl.BlockSpec((128, 512), lambda i: (0, 0)),
            pl.BlockSpec((1, 512), lambda i: (0, 0)),
        ],
        out_specs=pl.BlockSpec((blk, 256), lambda i: (i, 0)),
        out_shape=jax.ShapeDtypeStruct((NPAD3, 256), jnp.float32),
    )(acc0, acc1, den3, Wc_, biasc_flat)


def kernel(in_feat, edge_index, emb, W_src1, W_dst1, attn1, bias1,
           W_src2, W_dst2, attn2, bias2, Wc, attn_lc, attn_rc, biasc):
    src = edge_index[0].astype(jnp.int32)
    dst = edge_index[1].astype(jnp.int32)
    srcp = jnp.concatenate([src, jnp.zeros((EPAD - E,), jnp.int32)])
    dstp = jnp.concatenate([dst, jnp.full((EPAD - E,), 1 << 28, jnp.int32)])
    inf3 = jnp.concatenate(
        [in_feat.astype(jnp.int32), jnp.zeros((NPAD - N,), jnp.int32)]
    ).reshape(NPAD // 256, 1, 256)

    fs1, fd1 = _prep(inf3, emb, W_src1, W_dst1)
    acc1, den1f = _edge_v2(fs1, fd1, srcp, dstp, attn1)
    den1 = den1f.reshape(NPAD, 2)
    fs2, fd2 = _mid(acc1, den1, bias1.reshape(1, 128), W_src2, W_dst2)
    acc2, den2f = _edge_v2(fs2, fd2, srcp, dstp, attn2)
    den2 = den2f.reshape(NPAD, 2)
    h2, elr, erp = _mid2(acc2, den2, bias2.reshape(1, 128), Wc,
                         attn_lc, attn_rc)
    acc30, acc31, den3f = _edge_gat(h2, elr, erp, srcp, dstp)
    den3 = den3f.reshape(NPAD3, 2)
    out = _final(acc30, acc31, den3, Wc, biasc.reshape(1, 512))
    return out[:N]


# speedup vs baseline: 36.4016x; 1.0322x over previous
import functools

import jax
import jax.numpy as jnp
from jax import lax
from jax.experimental import pallas as pl
from jax.experimental.pallas import tpu as pltpu
from jax.experimental.pallas import tpu_sc as plsc

N = 50000
E = 800000
VOCAB = 256
EMB = 64

NR = 8192
RPC = 4
NPAD = NR * RPC * 2
NR3 = 3584
RPC3 = 7
NPAD3 = NR3 * RPC3 * 2
CB = 2048
NCHUNK = 25
EPAD = 16 * NCHUNK * CB
B = 128
HB = 2304
ZR = 16


def _i16(v):
    return jnp.full((16,), v, jnp.int32)


def _mo(x, m=8):
    return pl.multiple_of(x, m)


def _make_edge_v2():
    mesh = plsc.VectorSubcoreMesh(core_axis_name="c", subcore_axis_name="s")

    @functools.partial(
        pl.kernel,
        mesh=mesh,
        compiler_params=pltpu.CompilerParams(needs_layout_passes=False),
        out_type=[
            jax.ShapeDtypeStruct((NPAD, 128), jnp.float32),
            jax.ShapeDtypeStruct((NPAD * 2 // 128, 128), jnp.float32),
        ],
        scratch_types=[
            pltpu.VMEM_SHARED((NR + 8, 128), jnp.float32),
            pltpu.VMEM_SHARED((128, 128), jnp.float32),
            pltpu.VMEM((CB,), jnp.int32),
            pltpu.VMEM((CB,), jnp.int32),
            pltpu.VMEM((HB,), jnp.int32),
            pltpu.VMEM((HB,), jnp.int32),
            pltpu.VMEM((B,), jnp.int32),
            pltpu.VMEM((B,), jnp.int32),
            pltpu.VMEM((B,), jnp.int32),
            pltpu.VMEM((B + 16,), jnp.int32),
            pltpu.VMEM((B,), jnp.int32),
            pltpu.VMEM((B, 128), jnp.float32),
            pltpu.VMEM((B, 128), jnp.float32),
            pltpu.VMEM((136, 128), jnp.float32),
            pltpu.VMEM((ZR, 128), jnp.float32),
            pltpu.VMEM((2, 64), jnp.float32),
            pltpu.SemaphoreType.DMA,
            pltpu.SemaphoreType.DMA,
        ],
    )
    def k(fs_hbm, fd_hbm, src_hbm, dst_hbm, attn_hbm, acc_hbm, den_hbm,
          acc_sp, den_sp, sbuf, dbuf, hs, hd, gsrc, gdst, locb, locpad,
          ident, fsrow, fdrow, den_tile, zblk, attn_v, sem, sem2):
        c = lax.axis_index("c")
        s = lax.axis_index("s")
        iota16 = lax.iota(jnp.int32, 16)
        zeros16 = jnp.zeros((16,), jnp.float32)
        ebase = s * (EPAD // 16)
        lane0 = iota16 == 0

        pltpu.sync_copy(attn_hbm, attn_v)

        def zrow(r, _):
            for kk in range(8):
                zblk[r, pl.ds(kk * 16, 16)] = zeros16
            return 0
        lax.fori_loop(0, ZR, zrow, 0)
        for g in range(8):
            ident[pl.ds(g * 16, 16)] = iota16 + g * 16

        def process_block(off, valid, lo):
            def bg(g, _):
                dv = hd[pl.ds(off + g * 16, 16)]
                sv = hs[pl.ds(off + g * 16, 16)]
                m = (g * 16 + iota16) < valid
                lc = jnp.where(m, dv - lo, NR)
                gdst[pl.ds(g * 16, 16)] = jnp.where(m, dv, 0)
                gsrc[pl.ds(g * 16, 16)] = jnp.where(m, sv, 0)
                locb[pl.ds(g * 16, 16)] = lc
                locpad[pl.ds(g * 16, 16)] = lc
                return 0
            lax.fori_loop(0, B // 16, bg, 0)
            cp1 = pltpu.async_copy(fs_hbm.at[gsrc], fsrow, sem)
            cp2 = pltpu.async_copy(fd_hbm.at[gdst], fdrow, sem2)
            cp1.wait()
            cp2.wait()
            av = tuple(attn_v[hh, pl.ds(kk * 16, 16)]
                       for hh in range(2) for kk in range(4))

            def pedge(p, carry):
                x = [fsrow[p, pl.ds(kk * 16, 16)] for kk in range(8)]
                y = [fdrow[p, pl.ds(kk * 16, 16)] for kk in range(8)]
                w = []
                for kk in range(8):
                    t = x[kk] + y[kk]
                    w.append(jnp.maximum(t, 0.2 * t) * carry[kk])
                l0 = (w[0] + w[1]) + (w[2] + w[3])
                l1 = (w[4] + w[5]) + (w[6] + w[7])
                for sh in (1, 2, 4, 8):
                    l0 = l0 + l0[iota16 ^ sh]
                    l1 = l1 + l1[iota16 ^ sh]
                ex0 = jnp.exp(l0)
                ex1 = jnp.exp(l1)
                for kk in range(4):
                    fsrow[p, pl.ds(kk * 16, 16)] = x[kk] * ex0
                for kk in range(4, 8):
                    fsrow[p, pl.ds(kk * 16, 16)] = x[kk] * ex1
                ls = locpad[pl.ds(p, 16)][_i16(0)]
                dm = ls == NR
                d0 = jnp.where(dm, _i16(2 * NR), 2 * ls)
                d1 = jnp.where(dm, _i16(2 * NR), 2 * ls + 1)
                plsc.addupdate_scatter(
                    den_tile, [d0 >> 7, d0 & 127], ex0, mask=lane0)
                plsc.addupdate_scatter(
                    den_tile, [d1 >> 7, d1 & 127], ex1, mask=lane0)
                return carry
            lax.fori_loop(0, B, pedge, av)
            pltpu.sync_copy(fsrow, acc_sp.at[locb], add=True)

        def sweep(r, _):
            lo = (2 * r + c) * NR

            def zcp(j, _):
                row0 = _mo(s * (NR // 16) + j * ZR)
                pltpu.sync_copy(zblk, acc_sp.at[pl.ds(row0, ZR), :])
                return 0
            lax.fori_loop(0, NR // 16 // ZR, zcp, 0)

            def zdt(j, _):
                for kk in range(8):
                    den_tile[j, pl.ds(kk * 16, 16)] = zeros16
                return 0
            lax.fori_loop(0, 136, zdt, 0)
            pltpu.sync_copy(zblk.at[pl.ds(0, 8), :],
                            den_sp.at[pl.ds(_mo(s * 8), 8), :])
            plsc.subcore_barrier()

            def chunk(kk, cnt):
                pltpu.sync_copy(src_hbm.at[pl.ds(_mo(ebase + kk * CB), CB)], sbuf)
                pltpu.sync_copy(dst_hbm.at[pl.ds(_mo(ebase + kk * CB), CB)], dbuf)

                def scan(g, cnt):
                    dv = dbuf[pl.ds(g * 16, 16)]
                    sv = sbuf[pl.ds(g * 16, 16)]
                    m = (dv >= lo) & (dv < lo + NR)
                    pos = plsc.cumsum(m.astype(jnp.int32)) - 1 + cnt
                    plsc.store_scatter(hd, [pos], dv, mask=m)
                    plsc.store_scatter(hs, [pos], sv, mask=m)
                    return cnt + plsc.all_reduce_population_count(m)
                cnt = lax.fori_loop(0, CB // 16, scan, cnt)
                navail = cnt[0]
                nblk = navail // B

                def pb(i, _):
                    process_block(i * B, B, lo)
                    return 0
                lax.fori_loop(0, nblk, pb, 0)
                rem = navail - nblk * B

                def mv(g, _):
                    @pl.when(g * 16 < rem)
                    def _():
                        hd[pl.ds(g * 16, 16)] = hd[pl.ds(nblk * B + g * 16, 16)]
                        hs[pl.ds(g * 16, 16)] = hs[pl.ds(nblk * B + g * 16, 16)]
                    return 0
                lax.fori_loop(0, B // 16, mv, 0)
                return _i16(0) + rem
            cnt = lax.fori_loop(0, NCHUNK, chunk, _i16(0))
            rem = cnt[0]

            @pl.when(rem > 0)
            def _():
                process_block(0, rem, lo)
            plsc.subcore_barrier()
            pltpu.sync_copy(den_tile.at[pl.ds(0, 128), :],
                            den_sp.at[ident], add=True)
            plsc.subcore_barrier()

            def fcp(j, _):
                row0 = _mo(s * (NR // 16) + j * ZR)
                pltpu.sync_copy(acc_sp.at[pl.ds(row0, ZR), :],
                                acc_hbm.at[pl.ds(_mo(lo + row0), ZR), :])
                return 0
            lax.fori_loop(0, NR // 16 // ZR, fcp, 0)

            @pl.when(s == 0)
            def _():
                pltpu.sync_copy(den_sp,
                                den_hbm.at[pl.ds(_mo(lo // 64), 128), :])
            plsc.subcore_barrier()
            return 0
        lax.fori_loop(0, RPC, sweep, 0)

    return k


def _make_edge_gat():
    mesh = plsc.VectorSubcoreMesh(core_axis_name="c", subcore_axis_name="s")

    @functools.partial(
        pl.kernel,
        mesh=mesh,
        compiler_params=pltpu.CompilerParams(needs_layout_passes=False),
        out_type=[
            jax.ShapeDtypeStruct((NPAD3, 128), jnp.float32),
            jax.ShapeDtypeStruct((NPAD3, 128), jnp.float32),
            jax.ShapeDtypeStruct((NPAD3 * 2 // 128, 128), jnp.float32),
        ],
        scratch_types=[
            pltpu.VMEM_SHARED((NR3 + 8, 128), jnp.float32),
            pltpu.VMEM_SHARED((NR3 + 8, 128), jnp.float32),
            pltpu.VMEM_SHARED((128, 128), jnp.float32),
            pltpu.VMEM((CB,), jnp.int32),
            pltpu.VMEM((CB,), jnp.int32),
            pltpu.VMEM((HB,), jnp.int32),
            pltpu.VMEM((HB,), jnp.int32),
            pltpu.VMEM((B,), jnp.int32),
            pltpu.VMEM((B,), jnp.int32),
            pltpu.VMEM((B,), jnp.int32),
            pltpu.VMEM((B + 16,), jnp.int32),
            pltpu.VMEM((B,), jnp.int32),
            pltpu.VMEM((B + 16,), jnp.float32),
            pltpu.VMEM((B, 128), jnp.float32),
            pltpu.VMEM((B, 128), jnp.float32),
            pltpu.VMEM((NR3 + 16,), jnp.float32),
            pltpu.VMEM((NR3 + 16,), jnp.float32),
            pltpu.VMEM((136, 128), jnp.float32),
            pltpu.VMEM((ZR, 128), jnp.float32),
            pltpu.SemaphoreType.DMA,
            pltpu.SemaphoreType.DMA,
        ],
    )
    def k(h2_hbm, elr_hbm, erp_hbm, src_hbm, dst_hbm, acc0_hbm, acc1_hbm,
          den_hbm, acc0_sp, acc1_sp, den_sp, sbuf, dbuf, hs, hd, gsrc, gdst,
          locb, locpad, ident, exrb, h2row, elrow, er0rng, er1rng, den_tile,
          zblk, sem, sem2):
        c = lax.axis_index("c")
        s = lax.axis_index("s")
        iota16 = lax.iota(jnp.int32, 16)
        zeros16 = jnp.zeros((16,), jnp.float32)
        ebase = s * (EPAD // 16)
        lane0 = iota16 == 0

        def zrow(r, _):
            for kk in range(8):
                zblk[r, pl.ds(kk * 16, 16)] = zeros16
            return 0
        lax.fori_loop(0, ZR, zrow, 0)
        for g in range(8):
            ident[pl.ds(g * 16, 16)] = iota16 + g * 16

        def process_block(off, valid, lo):
            def bg(g, _):
                dv = hd[pl.ds(off + g * 16, 16)]
                sv = hs[pl.ds(off + g * 16, 16)]
                m = (g * 16 + iota16) < valid
                lc = jnp.where(m, dv - lo, NR3)
                gdst[pl.ds(g * 16, 16)] = jnp.where(m, dv, 0)
                gsrc[pl.ds(g * 16, 16)] = jnp.where(m, sv, 0)
                locb[pl.ds(g * 16, 16)] = lc
                locpad[pl.ds(g * 16, 16)] = lc
                return 0
            lax.fori_loop(0, B // 16, bg, 0)
            cp1 = pltpu.async_copy(h2_hbm.at[gsrc], h2row, sem)
            cp2 = pltpu.async_copy(elr_hbm.at[gsrc], elrow, sem2)
            cp1.wait()
            cp2.wait()

            def pedge(p, _):
                evl = elrow[p, pl.ds(0, 16)]
                ls = locpad[pl.ds(p, 16)][_i16(0)]
                t0 = evl[_i16(0)] + plsc.load_gather(er0rng, [ls])
                t1 = evl[_i16(1)] + plsc.load_gather(er1rng, [ls])
                ex0 = jnp.exp(jnp.maximum(t0, 0.2 * t0))
                ex1 = jnp.exp(jnp.maximum(t1, 0.2 * t1))
                for kk in range(8):
                    v = h2row[p, pl.ds(kk * 16, 16)]
                    h2row[p, pl.ds(kk * 16, 16)] = v * ex0
                plsc.store_scatter(exrb, [_i16(0) + p], ex1 / ex0, mask=lane0)
                dm = ls == NR3
                d0 = jnp.where(dm, _i16(2 * NR3), 2 * ls)
                d1 = jnp.where(dm, _i16(2 * NR3), 2 * ls + 1)
                plsc.addupdate_scatter(
                    den_tile, [d0 >> 7, d0 & 127], ex0, mask=lane0)
                plsc.addupdate_scatter(
                    den_tile, [d1 >> 7, d1 & 127], ex1, mask=lane0)
                return 0
            lax.fori_loop(0, B, pedge, 0)
            pltpu.sync_copy(h2row, acc0_sp.at[locb], add=True)

            def redge(p, _):
                rs = exrb[pl.ds(p, 16)][_i16(0)]
                for kk in range(8):
                    h2row[p, pl.ds(kk * 16, 16)] = (
                        h2row[p, pl.ds(kk * 16, 16)] * rs)
                return 0
            lax.fori_loop(0, B, redge, 0)
            pltpu.sync_copy(h2row, acc1_sp.at[locb], add=True)

        def sweep(r, _):
            lo = (2 * r + c) * NR3
            pltpu.sync_copy(erp_hbm.at[0, pl.ds(lo, NR3)],
                            er0rng.at[pl.ds(0, NR3)])
            pltpu.sync_copy(erp_hbm.at[1, pl.ds(lo, NR3)],
                            er1rng.at[pl.ds(0, NR3)])

            def zcp(j, _):
                row0 = _mo(s * (NR3 // 16) + j * ZR)
                pltpu.sync_copy(zblk, acc0_sp.at[pl.ds(row0, ZR), :])
                pltpu.sync_copy(zblk, acc1_sp.at[pl.ds(row0, ZR), :])
                return 0
            lax.fori_loop(0, NR3 // 16 // ZR, zcp, 0)

            def zdt(j, _):
                for kk in range(8):
                    den_tile[j, pl.ds(kk * 16, 16)] = zeros16
                return 0
            lax.fori_loop(0, 136, zdt, 0)
            pltpu.sync_copy(zblk.at[pl.ds(0, 8), :],
                            den_sp.at[pl.ds(_mo(s * 8), 8), :])
            plsc.subcore_barrier()

            def chunk(kk, cnt):
                pltpu.sync_copy(src_hbm.at[pl.ds(_mo(ebase + kk * CB), CB)], sbuf)
                pltpu.sync_copy(dst_hbm.at[pl.ds(_mo(ebase + kk * CB), CB)], dbuf)

                def scan(g, cnt):
                    dv = dbuf[pl.ds(g * 16, 16)]
                    sv = sbuf[pl.ds(g * 16, 16)]
                    m = (dv >= lo) & (dv < lo + NR3)
                    pos = plsc.cumsum(m.astype(jnp.int32)) - 1 + cnt
                    plsc.store_scatter(hd, [pos], dv, mask=m)
                    plsc.store_scatter(hs, [pos], sv, mask=m)
                    return cnt + plsc.all_reduce_population_count(m)
                cnt = lax.fori_loop(0, CB // 16, scan, cnt)
                navail = cnt[0]
                nblk = navail // B

                def pb(i, _):
                    process_block(i * B, B, lo)
                    return 0
                lax.fori_loop(0, nblk, pb, 0)
                rem = navail - nblk * B

                def mv(g, _):
                    @pl.when(g * 16 < rem)
                    def _():
                        hd[pl.ds(g * 16, 16)] = hd[pl.ds(nblk * B + g * 16, 16)]
                        hs[pl.ds(g * 16, 16)] = hs[pl.ds(nblk * B + g * 16, 16)]
                    return 0
                lax.fori_loop(0, B // 16, mv, 0)
                return _i16(0) + rem
            cnt = lax.fori_loop(0, NCHUNK, chunk, _i16(0))
            rem = cnt[0]

            @pl.when(rem > 0)
            def _():
                process_block(0, rem, lo)
            plsc.subcore_barrier()
            pltpu.sync_copy(den_tile.at[pl.ds(0, 128), :],
                            den_sp.at[ident], add=True)
            plsc.subcore_barrier()

            def fcp(j, _):
                row0 = _mo(s * (NR3 // 16) + j * ZR)
                pltpu.sync_copy(acc0_sp.at[pl.ds(row0, ZR), :],
                                acc0_hbm.at[pl.ds(_mo(lo + row0), ZR), :])
                pltpu.sync_copy(acc1_sp.at[pl.ds(row0, ZR), :],
                                acc1_hbm.at[pl.ds(_mo(lo + row0), ZR), :])
                return 0
            lax.fori_loop(0, NR3 // 16 // ZR, fcp, 0)

            @pl.when(s == 0)
            def _():
                pltpu.sync_copy(den_sp.at[pl.ds(0, 56), :],
                                den_hbm.at[pl.ds(_mo(lo // 64), 56), :])
            plsc.subcore_barrier()
            return 0
        lax.fori_loop(0, RPC3, sweep, 0)

    return k


_edge_v2 = _make_edge_v2()
_edge_gat = _make_edge_gat()


def _prep(in_feat3, emb, Ws, Wd):
    def body(if_ref, emb_ref, ws_ref, wd_ref, fs_ref, fd_ref):
        ids = if_ref[0, 0, :]
        oh = (ids[:, None] == lax.broadcasted_iota(jnp.int32, (256, VOCAB), 1)
              ).astype(jnp.float32)
        h = jnp.dot(oh, emb_ref[...], preferred_element_type=jnp.float32)
        fs_ref[...] = jnp.dot(h, ws_ref[...], preferred_element_type=jnp.float32)
        fd_ref[...] = jnp.dot(h, wd_ref[...], preferred_element_type=jnp.float32)

    return pl.pallas_call(
        body,
        grid=(NPAD // 256,),
        in_specs=[
            pl.BlockSpec((1, 1, 256), lambda i: (i, 0, 0)),
            pl.BlockSpec((VOCAB, EMB), lambda i: (0, 0)),
            pl.BlockSpec((EMB, 128), lambda i: (0, 0)),
            pl.BlockSpec((EMB, 128), lambda i: (0, 0)),
        ],
        out_specs=[
            pl.BlockSpec((256, 128), lambda i: (i, 0)),
            pl.BlockSpec((256, 128), lambda i: (i, 0)),
        ],
        out_shape=[jax.ShapeDtypeStruct((NPAD, 128), jnp.float32)] * 2,
    )(in_feat3, emb, Ws, Wd)


def _normalize(a_ref, d_ref, b_ref):
    den0 = d_ref[:, 0:1] + 1e-16
    den1 = d_ref[:, 1:2] + 1e-16
    h = jnp.concatenate([a_ref[:, :64] / den0, a_ref[:, 64:] / den1], axis=1)
    h = h + b_ref[...]
    return jnp.where(h > 0, h, jnp.exp(jnp.minimum(h, 0.0)) - 1.0)


def _mid(acc, den, bias_flat, Wa, Wb):
    blk = 1024

    def body(a_ref, d_ref, b_ref, wa_ref, wb_ref, fs_ref, fd_ref):
        h = _normalize(a_ref, d_ref, b_ref)
        fs_ref[...] = jnp.dot(h, wa_ref[...], preferred_element_type=jnp.float32)
        fd_ref[...] = jnp.dot(h, wb_ref[...], preferred_element_type=jnp.float32)

    return pl.pallas_call(
        body,
        grid=(NPAD // blk,),
        in_specs=[
            pl.BlockSpec((blk, 128), lambda i: (i, 0)),
            pl.BlockSpec((blk, 2), lambda i: (i, 0)),
            pl.BlockSpec((1, 128), lambda i: (0, 0)),
            pl.BlockSpec((128, 128), lambda i: (0, 0)),
            pl.BlockSpec((128, 128), lambda i: (0, 0)),
        ],
        out_specs=[
            pl.BlockSpec((blk, 128), lambda i: (i, 0)),
            pl.BlockSpec((blk, 128), lambda i: (i, 0)),
        ],
        out_shape=[jax.ShapeDtypeStruct((NPAD, 128), jnp.float32)] * 2,
    )(acc, den, bias_flat, Wa, Wb)


def _mid2(acc, den, bias_flat, Wc_, al, ar):
    blk = 1024

    def body(a_ref, d_ref, b_ref, wc_ref, al_ref, ar_ref, h_ref, elr_ref,
             erp_ref):
        h = _normalize(a_ref, d_ref, b_ref)
        h_ref[...] = h
        f = jnp.dot(h, wc_ref[...], preferred_element_type=jnp.float32)
        el0 = jnp.sum(f[:, :256] * al_ref[0, :][None, :], axis=1, keepdims=True)
        el1 = jnp.sum(f[:, 256:] * al_ref[1, :][None, :], axis=1, keepdims=True)
        elr_ref[...] = jnp.concatenate(
            [el0, el1, jnp.zeros((blk, 126), jnp.float32)], axis=1)
        dn = (((1,), (1,)), ((), ()))
        wr0 = lax.dot_general(ar_ref[0:1, :], wc_ref[:, :256], dn,
                              preferred_element_type=jnp.float32)
        wr1 = lax.dot_general(ar_ref[1:2, :], wc_ref[:, 256:], dn,
                              preferred_element_type=jnp.float32)
        er0 = lax.dot_general(wr0, h, dn, preferred_element_type=jnp.float32)
        er1 = lax.dot_general(wr1, h, dn, preferred_element_type=jnp.float32)
        erp_ref[...] = jnp.concatenate([er0, er1], axis=0)

    return pl.pallas_call(
        body,
        grid=(NPAD // blk,),
        in_specs=[
            pl.BlockSpec((blk, 128), lambda i: (i, 0)),
            pl.BlockSpec((blk, 2), lambda i: (i, 0)),
            pl.BlockSpec((1, 128), lambda i: (0, 0)),
            pl.BlockSpec((128, 512), lambda i: (0, 0)),
            pl.BlockSpec((2, 256), lambda i: (0, 0)),
            pl.BlockSpec((2, 256), lambda i: (0, 0)),
        ],
        out_specs=[
            pl.BlockSpec((blk, 128), lambda i: (i, 0)),
            pl.BlockSpec((blk, 128), lambda i: (i, 0)),
            pl.BlockSpec((2, blk), lambda i: (0, i)),
        ],
        out_shape=[
            jax.ShapeDtypeStruct((NPAD, 128), jnp.float32),
            jax.ShapeDtypeStruct((NPAD, 128), jnp.float32),
            jax.ShapeDtypeStruct((2, NPAD), jnp.float32),
        ],
    )(acc, den, bias_flat, Wc_, al, ar)


def _final(acc0, acc1, den3, Wc_, biasc_flat):
    blk = 1024

    def body(a0_ref, a1_ref, d_ref, wc_ref, b_ref, o_ref):
        g0 = a0_ref[...] / (d_ref[:, 0:1] + 1e-16)
        g1 = a1_ref[...] / (d_ref[:, 1:2] + 1e-16)
        o = (jnp.dot(g0, wc_ref[:, :256], preferred_element_type=jnp.float32)
             + jnp.dot(g1, wc_ref[:, 256:], preferred_element_type=jnp.float32)
             + b_ref[0, :256] + b_ref[0, 256:])
        o = o * 0.5
        m = jnp.max(o, axis=1, keepdims=True)
        ex = jnp.exp(o - m)
        o_ref[...] = ex / jnp.sum(ex, axis=1, keepdims=True)

    return pl.pallas_call(
        body,
        grid=(NPAD3 // blk,),
        in_specs=[
            pl.BlockSpec((blk, 128), lambda i: (i, 0)),
            pl.BlockSpec((blk, 128), lambda i: (i, 0)),
            pl.BlockSpec((blk, 2), lambda i: (i, 0)),
            pl.BlockSpec((128, 512), lambda i: (0, 0)),
            pl.BlockSpec((1, 512), lambda i: (0, 0)),
        ],
        out_specs=pl.BlockSpec((blk, 256), lambda i: (i, 0)),
        out_shape=jax.ShapeDtypeStruct((NPAD3, 256), jnp.float32),
    )(acc0, acc1, den3, Wc_, biasc_flat)


def kernel(in_feat, edge_index, emb, W_src1, W_dst1, attn1, bias1,
           W_src2, W_dst2, attn2, bias2, Wc, attn_lc, attn_rc, biasc):
    src = edge_index[0].astype(jnp.int32)
    dst = edge_index[1].astype(jnp.int32)
    srcp = jnp.concatenate([src, jnp.zeros((EPAD - E,), jnp.int32)])
    dstp = jnp.concatenate([dst, jnp.full((EPAD - E,), 1 << 28, jnp.int32)])
    inf3 = jnp.concatenate(
        [in_feat.astype(jnp.int32), jnp.zeros((NPAD - N,), jnp.int32)]
    ).reshape(NPAD // 256, 1, 256)

    fs1, fd1 = _prep(inf3, emb, W_src1, W_dst1)
    acc1, den1f = _edge_v2(fs1, fd1, srcp, dstp, attn1)
    den1 = den1f.reshape(NPAD, 2)
    fs2, fd2 = _mid(acc1, den1, bias1.reshape(1, 128), W_src2, W_dst2)
    acc2, den2f = _edge_v2(fs2, fd2, srcp, dstp, attn2)
    den2 = den2f.reshape(NPAD, 2)
    h2, elr, erp = _mid2(acc2, den2, bias2.reshape(1, 128), Wc,
                         attn_lc, attn_rc)
    acc30, acc31, den3f = _edge_gat(h2, elr, erp, srcp, dstp)
    den3 = den3f.reshape(NPAD3, 2)
    out = _final(acc30, acc31, den3, Wc, biasc.reshape(1, 512))
    return out[:N]
